# Initial kernel scaffold; baseline (speedup 1.0000x reference)
import jax
import jax.numpy as jnp
from jax.experimental import pallas as pl


def kernel(x, mask, A_in_pick, A_in_src, A_src_in_product, A_sta_in_product, A_src_in_sta, locs_cart, srcs_cart, memory, params):
    raise NotImplementedError("write your pallas kernel here")


# speedup vs baseline: 1.0011x; 1.0011x over previous
import jax
import jax.numpy as jnp
from jax.experimental import pallas as pl

N_PROD = 50000
N_STA = 500
N_SRC = 1000


def _prelu(x, a):
    return jnp.where(x >= 0, x, a * x)


def _lin(x, p):
    return x @ p['W'].T + p['b']


def _mlp(x, p):
    h = _prelu(_lin(x, p['layers'][0]), p['a'])
    return _lin(h, p['layers'][1])


def _seg_mean(m, dst, num):
    s = jax.ops.segment_sum(m, dst, num_segments=num)
    c = jax.ops.segment_sum(jnp.ones((m.shape[0], 1), m.dtype), dst, num_segments=num)
    return s / jnp.maximum(c, 1.0)


def _copy_kernel(x_ref, o_ref):
    o_ref[...] = x_ref[...]


def _pl_copy(x):
    return pl.pallas_call(
        _copy_kernel,
        out_shape=jax.ShapeDtypeStruct(x.shape, x.dtype),
    )(x)


def _data_agg(p, x_in, mask, A_in_sta, A_in_src, A_src_in_sta, pos_loc, pos_src):
    N = x_in.shape[0]
    tr = jnp.concatenate([x_in, mask], axis=-1)
    tr = _prelu(_lin(tr, p['init_trns']), p['a_init'])
    pos_rel_sta = (pos_loc[A_src_in_sta[0][A_in_sta[0]]] / 1000.0 - pos_loc[A_src_in_sta[0][A_in_sta[1]]] / 1000.0) / 30.0
    pos_rel_src = (pos_src[A_src_in_sta[1][A_in_src[0]]] / 1000.0 - pos_src[A_src_in_sta[1][A_in_src[1]]] / 1000.0) / 30.0

    def prop(h, edges, edge_attr):
        m = jnp.concatenate([h[edges[0]], edge_attr], axis=1)
        m = _prelu(_lin(m, p['merge_edges']), p['a_merge'])
        return _seg_mean(m, edges[1], N)

    tr1 = _lin(jnp.concatenate([tr, prop(_prelu(tr, p['a11']), A_in_sta, pos_rel_sta), mask], axis=1), p['l1_t1_2'])
    tr2 = _lin(jnp.concatenate([tr, prop(_prelu(tr, p['a12']), A_in_src, pos_rel_src), mask], axis=1), p['l1_t2_2'])
    tr = _prelu(jnp.concatenate([tr1, tr2], axis=1), p['a1'])
    tr1 = _lin(jnp.concatenate([tr, prop(_prelu(_lin(tr, p['l2_t1_1']), p['a21']), A_in_sta, pos_rel_sta), mask], axis=1), p['l2_t1_2'])
    tr2 = _lin(jnp.concatenate([tr, prop(_prelu(_lin(tr, p['l2_t2_1']), p['a22']), A_in_src, pos_rel_src), mask], axis=1), p['l2_t2_2'])
    return _prelu(jnp.concatenate([tr1, tr2], axis=1), p['a2'])


def _bip_readout(p, x, mask, edges, pos_src_side, pos_dst_side, M):
    xm = jnp.concatenate([x, mask], axis=1)
    pos_j = pos_src_side[edges[0]]
    pos_i = pos_dst_side[edges[1]]
    m = jnp.concatenate([xm[edges[0]], (pos_i - pos_j) / 30000.0], axis=1)
    m = _lin(m, p['fc1_0'])
    m = _prelu(m, p['a_fc1'])
    m = _lin(m, p['fc1_2'])
    m = _prelu(m, p['a1'])
    agg = _seg_mean(m, edges[1], M)
    return _prelu(_lin(agg, p['fc2']), p['a2'])


def kernel(x, mask, A_in_pick, A_in_src, A_src_in_product, A_sta_in_product, A_src_in_sta, locs_cart, srcs_cart, memory, params):
    mem_g = memory[A_src_in_sta[1]]
    mask_e = _mlp(jnp.concatenate([mask, mem_g], axis=1), params['embed_inpt'])
    h = jnp.concatenate([x, mem_g], axis=1)
    h = _pl_copy(h)
    for name in ['da1', 'da2', 'da3', 'da4', 'da5']:
        h = _data_agg(params[name], h, mask_e, A_in_pick, A_in_src, A_src_in_sta, locs_cart, srcs_cart)
    src_emb = _bip_readout(params['bip_src'], h, mask_e, A_src_in_product, locs_cart[A_src_in_sta[0]], srcs_cart, N_SRC)
    mem_p = _mlp(memory, params['proj_memory'])
    src_emb = _mlp(jnp.concatenate([src_emb, mem_p], axis=1), params['merge_data'])
    pred = _mlp(src_emb, params['proj']) * 5000.0
    pred_t = _mlp(src_emb, params['proj_t'])
    sta_emb = _bip_readout(params['bip_sta'], h, mask_e, A_sta_in_product, srcs_cart[A_src_in_sta[1]], locs_cart, N_STA)
    corr = _mlp(sta_emb, params['proj_c'])
    return pred, pred_t, corr


# speedup vs baseline: 8.4105x; 8.4009x over previous
import functools

import jax
import jax.numpy as jnp
from jax import lax
from jax.experimental import pallas as pl
from jax.experimental.pallas import tpu as pltpu
from jax.experimental.pallas import tpu_sc as plsc

NREAL = 50000
NPAD = 53248
F = 32
FT = 16
NE = 1600000
WSZ = 128
NWRK = 32
NC = 2
NSUB = 16
NW_E = 391
EPAD = NWRK * NW_E * WSZ
NW_R = 13
NSEG_SRC = 1024
NSEG_STA = 512
BM = 512
GRID = NPAD // BM

_f32 = jnp.float32


def _mesh():
    return plsc.VectorSubcoreMesh(core_axis_name="c", subcore_axis_name="s")


def _make_prop():
    @functools.partial(
        pl.kernel,
        compiler_params=pltpu.CompilerParams(use_tc_tiling_on_sc=False),
        out_type=jax.ShapeDtypeStruct((NC, NPAD, F), _f32),
        mesh=_mesh(),
        scratch_types=[
            pltpu.VMEM((2, WSZ), jnp.int32),
            pltpu.VMEM((2, WSZ), jnp.int32),
            pltpu.VMEM((WSZ, F), _f32),
            pltpu.VMEM((WSZ, F), _f32),
            pltpu.VMEM((WSZ, F), _f32),
            pltpu.VMEM((WSZ, F), _f32),
            pltpu.VMEM((16,), _f32),
            pltpu.VMEM((WSZ, F), _f32),
            pltpu.VMEM_SHARED((NPAD, F), _f32),
            pltpu.SemaphoreType.DMA,
            pltpu.SemaphoreType.DMA,
            pltpu.SemaphoreType.DMA,
            pltpu.SemaphoreType.DMA,
        ],
    )
    def prop(a_hbm, g_hbm, idx_hbm, alpha_hbm, out_hbm,
             idx0, idx1, a0, a1, g0, g1, alphav, zbuf, acc,
             sa0, sa1, sg0, sg1):
        c = lax.axis_index("c")
        s = lax.axis_index("s")
        wid = s * NC + c
        rows_per_sub = NPAD // NSUB
        rows0 = s * rows_per_sub

        def zb(i, _):
            zbuf[i, pl.ds(0, 16)] = jnp.zeros((16,), _f32)
            zbuf[i, pl.ds(16, 16)] = jnp.zeros((16,), _f32)
            return 0
        lax.fori_loop(0, WSZ, zb, 0)

        def zc(k, _):
            pltpu.sync_copy(zbuf, acc.at[pl.ds(rows0 + k * WSZ, WSZ)])
            return 0
        lax.fori_loop(0, rows_per_sub // WSZ, zc, 0)

        pltpu.sync_copy(alpha_hbm, alphav)
        av = alphav[...]
        plsc.subcore_barrier()

        wbase = wid * NW_E

        def load_idx(ibuf, w):
            pltpu.sync_copy(idx_hbm.at[wbase + w], ibuf)

        def fire(ibuf, ab, gb, sa, sg):
            pltpu.async_copy(a_hbm.at[ibuf.at[0]], ab, sa)
            pltpu.async_copy(g_hbm.at[ibuf.at[1]], gb, sg)

        def drain(ibuf, ab, gb, sa, sg):
            pltpu.make_async_copy(a_hbm.at[ibuf.at[0]], ab, sa).wait()
            pltpu.make_async_copy(g_hbm.at[ibuf.at[1]], gb, sg).wait()

        def compute(ab, gb):
            def cb(i, _):
                for hh in (0, 16):
                    d = ab[i, pl.ds(hh, 16)] - gb[i, pl.ds(hh, 16)]
                    ab[i, pl.ds(hh, 16)] = (jnp.maximum(d, 0.0)
                                            + av * jnp.minimum(d, 0.0))
                return 0
            lax.fori_loop(0, WSZ, cb, 0, unroll=4)

        def scat(ibuf, ab):
            pltpu.sync_copy(ab, acc.at[ibuf.at[1]], add=True)

        load_idx(idx0, 0)
        fire(idx0, a0, g0, sa0, sg0)

        def body(j, _):
            w0 = 2 * j

            @pl.when(w0 + 1 < NW_E)
            def _():
                load_idx(idx1, w0 + 1)
                fire(idx1, a1, g1, sa1, sg1)

            drain(idx0, a0, g0, sa0, sg0)
            compute(a0, g0)
            scat(idx0, a0)

            @pl.when(w0 + 2 < NW_E)
            def _():
                load_idx(idx0, w0 + 2)
                fire(idx0, a0, g0, sa0, sg0)

            @pl.when(w0 + 1 < NW_E)
            def _():
                drain(idx1, a1, g1, sa1, sg1)
                compute(a1, g1)
                scat(idx1, a1)
            return 0
        lax.fori_loop(0, (NW_E + 1) // 2, body, 0)

        plsc.subcore_barrier()
        pltpu.sync_copy(acc.at[pl.ds(rows0, rows_per_sub)],
                        out_hbm.at[c, pl.ds(rows0, rows_per_sub)])

    return prop


def _make_counts(nseg, nwin):
    rows_per_sub = nseg // NSUB

    @functools.partial(
        pl.kernel,
        compiler_params=pltpu.CompilerParams(use_tc_tiling_on_sc=False),
        out_type=jax.ShapeDtypeStruct((NC, nseg), _f32),
        mesh=_mesh(),
        scratch_types=[
            pltpu.VMEM((2, WSZ), jnp.int32),
            pltpu.VMEM((WSZ,), _f32),
            pltpu.VMEM_SHARED((nseg,), _f32),
        ],
    )
    def counts(idx_hbm, out_hbm, idxb, ones, acc):
        c = lax.axis_index("c")
        s = lax.axis_index("s")
        wid = s * NC + c
        rows0 = s * rows_per_sub

        def zb(i, _):
            ones[pl.ds(i * 16, 16)] = jnp.zeros((16,), _f32)
            return 0
        lax.fori_loop(0, WSZ // 16, zb, 0)
        if rows_per_sub <= WSZ:
            pltpu.sync_copy(ones.at[pl.ds(0, rows_per_sub)],
                            acc.at[pl.ds(rows0, rows_per_sub)])
        else:
            def zc(k, _):
                pltpu.sync_copy(ones, acc.at[pl.ds(rows0 + k * WSZ, WSZ)])
                return 0
            lax.fori_loop(0, rows_per_sub // WSZ, zc, 0)

        def ob(i, _):
            ones[pl.ds(i * 16, 16)] = jnp.full((16,), 1.0, _f32)
            return 0
        lax.fori_loop(0, WSZ // 16, ob, 0)
        plsc.subcore_barrier()

        wbase = wid * nwin

        def body(w, _):
            pltpu.sync_copy(idx_hbm.at[wbase + w], idxb)
            pltpu.sync_copy(ones, acc.at[idxb.at[1]], add=True)
            return 0
        lax.fori_loop(0, nwin, body, 0)

        plsc.subcore_barrier()
        pltpu.sync_copy(acc.at[pl.ds(rows0, rows_per_sub)],
                        out_hbm.at[c, pl.ds(rows0, rows_per_sub)])

    return counts


def _make_tab(ntab):
    @functools.partial(
        pl.kernel,
        compiler_params=pltpu.CompilerParams(use_tc_tiling_on_sc=False),
        out_type=jax.ShapeDtypeStruct((NPAD, FT), _f32),
        mesh=_mesh(),
        scratch_types=[
            pltpu.VMEM((WSZ,), jnp.int32),
            pltpu.VMEM((WSZ, FT), _f32),
            pltpu.SemaphoreType.DMA,
        ],
    )
    def tab(t_hbm, idx_hbm, out_hbm, idxb, tbuf, sem):
        c = lax.axis_index("c")
        s = lax.axis_index("s")
        wid = s * NC + c
        ebase = wid * NW_R * WSZ

        def body(w, _):
            pltpu.sync_copy(idx_hbm.at[wid * NW_R + w], idxb)
            pltpu.async_copy(t_hbm.at[idxb], tbuf, sem).wait()
            pltpu.sync_copy(tbuf, out_hbm.at[pl.ds(ebase + w * WSZ, WSZ)])
            return 0
        lax.fori_loop(0, NW_R, body, 0)

    return tab


def _make_emap(nseg):
    @functools.partial(
        pl.kernel,
        compiler_params=pltpu.CompilerParams(use_tc_tiling_on_sc=False),
        out_type=jax.ShapeDtypeStruct((NPAD, F), _f32),
        mesh=_mesh(),
        scratch_types=[
            pltpu.VMEM((2, WSZ), jnp.int32),
            pltpu.VMEM((WSZ, F), _f32),
            pltpu.VMEM((WSZ, F), _f32),
            pltpu.VMEM((16,), _f32),
            pltpu.SemaphoreType.DMA,
            pltpu.SemaphoreType.DMA,
        ],
    )
    def emap(c_hbm, d_hbm, idx_hbm, alpha_hbm, out_hbm,
             idxb, cbuf, dbuf, alphav, sc_, sd_):
        c = lax.axis_index("c")
        s = lax.axis_index("s")
        wid = s * NC + c
        ebase = wid * NW_R * WSZ
        pltpu.sync_copy(alpha_hbm, alphav)
        av = alphav[...]

        def body(w, _):
            pltpu.sync_copy(idx_hbm.at[wid * NW_R + w], idxb)
            pltpu.async_copy(c_hbm.at[idxb.at[0]], cbuf, sc_)
            pltpu.async_copy(d_hbm.at[idxb.at[1]], dbuf, sd_)
            pltpu.make_async_copy(c_hbm.at[idxb.at[0]], cbuf, sc_).wait()
            pltpu.make_async_copy(d_hbm.at[idxb.at[1]], dbuf, sd_).wait()

            def cb(i, _):
                for hh in (0, 16):
                    d = cbuf[i, pl.ds(hh, 16)] + dbuf[i, pl.ds(hh, 16)]
                    cbuf[i, pl.ds(hh, 16)] = (jnp.maximum(d, 0.0)
                                              + av * jnp.minimum(d, 0.0))
                return 0
            lax.fori_loop(0, WSZ, cb, 0, unroll=4)
            pltpu.sync_copy(cbuf, out_hbm.at[pl.ds(ebase + w * WSZ, WSZ)])
            return 0
        lax.fori_loop(0, NW_R, body, 0)

    return emap


def _make_rscatter(nseg):
    rows_per_sub = nseg // NSUB

    @functools.partial(
        pl.kernel,
        compiler_params=pltpu.CompilerParams(use_tc_tiling_on_sc=False),
        out_type=jax.ShapeDtypeStruct((NC, nseg, F), _f32),
        mesh=_mesh(),
        scratch_types=[
            pltpu.VMEM((2, WSZ), jnp.int32),
            pltpu.VMEM((WSZ, F), _f32),
            pltpu.VMEM((WSZ, F), _f32),
            pltpu.VMEM_SHARED((nseg, F), _f32),
            pltpu.SemaphoreType.DMA,
        ],
    )
    def rscatter(m_hbm, idx_hbm, out_hbm, idxb, mbuf, zbuf, acc, sem):
        c = lax.axis_index("c")
        s = lax.axis_index("s")
        wid = s * NC + c
        rows0 = s * rows_per_sub
        ebase = wid * NW_R * WSZ

        def zb(i, _):
            zbuf[i, pl.ds(0, 16)] = jnp.zeros((16,), _f32)
            zbuf[i, pl.ds(16, 16)] = jnp.zeros((16,), _f32)
            return 0
        lax.fori_loop(0, WSZ, zb, 0)
        pltpu.sync_copy(zbuf.at[pl.ds(0, rows_per_sub)],
                        acc.at[pl.ds(rows0, rows_per_sub)])
        plsc.subcore_barrier()

        def body(w, _):
            pltpu.sync_copy(idx_hbm.at[wid * NW_R + w], idxb)
            pltpu.async_copy(m_hbm.at[pl.ds(ebase + w * WSZ, WSZ)],
                             mbuf, sem).wait()
            pltpu.sync_copy(mbuf, acc.at[idxb.at[1]], add=True)
            return 0
        lax.fori_loop(0, NW_R, body, 0)

        plsc.subcore_barrier()
        pltpu.sync_copy(acc.at[pl.ds(rows0, rows_per_sub)],
                        out_hbm.at[c, pl.ds(rows0, rows_per_sub)])

    return rscatter


_PROP = _make_prop()
_COUNTS_BIG = _make_counts(NPAD, NW_E)
_COUNTS_SRC = _make_counts(NSEG_SRC, NW_R)
_COUNTS_STA = _make_counts(NSEG_STA, NW_R)
_TAB_SRC = _make_tab(NSEG_SRC)
_TAB_STA = _make_tab(NSEG_STA)
_EMAP_SRC = _make_emap(NSEG_SRC)
_EMAP_STA = _make_emap(NSEG_STA)
_RSCAT_SRC = _make_rscatter(NSEG_SRC)
_RSCAT_STA = _make_rscatter(NSEG_STA)


def _prelu(x, a):
    return jnp.where(x >= 0, x, a * x)


def _row_spec(f):
    return pl.BlockSpec((BM, f), lambda i: (i, 0))


def _stack_spec(f):
    return pl.BlockSpec((2, BM, f), lambda i: (0, i, 0))


def _full_spec(shape):
    nd = len(shape)
    return pl.BlockSpec(shape, lambda i: (0,) * nd)


def _smem_spec():
    return pl.BlockSpec(memory_space=pltpu.SMEM)


def _s0_body(al, xp, mk, tg, e0m, e0g, b0, e1, b1, sx, sg, me_o, h0_o):
    a_e = al[0]
    t8 = tg[...][:, 0:8]
    h = _prelu(mk[...] @ e0m[...].T + t8 @ e0g[...].T + b0[...], a_e)
    me_o[...] = h @ e1[...].T + b1[...]
    h0_o[...] = xp[...] @ sx[...].T + t8 @ sg[...].T


def _s1_body(al, h, me, psta, psrc, wih, wim, bi, wh, bme, we3,
             tr_o, a1_o, a2_o, g1_o, g2_o):
    a_init, a11, a12 = al[0], al[1], al[2]
    tr = _prelu(h[...] @ wih[...].T + me[...] @ wim[...].T + bi[...], a_init)
    g1 = psta[...] @ we3[...].T
    g2 = psrc[...] @ we3[...].T
    tr_o[...] = tr
    a1_o[...] = _prelu(tr, a11) @ wh[...].T + bme[...] + g1
    a2_o[...] = _prelu(tr, a12) @ wh[...].T + bme[...] + g2
    g1_o[...] = g1
    g2_o[...] = g2


def _s2_body(al, tr, me, p1, p2, c1, c2, g1, g2,
             wt, wp1, wp2, wm, bb, w21, b21, w22, b22, wh, bme,
             trp_o, a3_o, a4_o):
    a1s, a21, a22 = al[0], al[1], al[2]
    ic1 = 1.0 / jnp.maximum(c1[...][0] + c1[...][1], 1.0)
    ic2 = 1.0 / jnp.maximum(c2[...][0] + c2[...][1], 1.0)
    prop1 = (p1[...][0] + p1[...][1]) * ic1
    prop2 = (p2[...][0] + p2[...][1]) * ic2
    trp = _prelu(tr[...] @ wt[...].T + prop1 @ wp1[...].T
                 + prop2 @ wp2[...].T + me[...] @ wm[...].T + bb[...], a1s)
    u1 = _prelu(trp @ w21[...].T + b21[...], a21)
    u2 = _prelu(trp @ w22[...].T + b22[...], a22)
    trp_o[...] = trp
    a3_o[...] = u1 @ wh[...].T + bme[...] + g1[...]
    a4_o[...] = u2 @ wh[...].T + bme[...] + g2[...]


def _s3_body(al, trp, me, p3, p4, c1, c2, wt, wp1, wp2, wm, bb, h_o):
    a2s = al[0]
    ic1 = 1.0 / jnp.maximum(c1[...][0] + c1[...][1], 1.0)
    ic2 = 1.0 / jnp.maximum(c2[...][0] + c2[...][1], 1.0)
    prop3 = (p3[...][0] + p3[...][1]) * ic1
    prop4 = (p4[...][0] + p4[...][1]) * ic2
    h_o[...] = _prelu(trp[...] @ wt[...].T + prop3 @ wp1[...].T
                      + prop4 @ wp2[...].T + me[...] @ wm[...].T + bb[...], a2s)


def _sr1_body(h, me, psta, psrc, v1, v2, v3, u1, u2, u3, cs_o, ct_o):
    cs_o[...] = h[...] @ v1[...].T + me[...] @ v2[...].T - psta[...] @ v3[...].T
    ct_o[...] = h[...] @ u1[...].T + me[...] @ u2[...].T - psrc[...] @ u3[...].T


def _sr2_body(al, m1, w, b, o):
    o[...] = _prelu(m1[...] @ w[...].T + b[...], al[0])


def _sd_body(srcsp, locsp, v3, b1s, u3, b1t, ds_o, dt_o):
    ds_o[...] = srcsp[...] @ v3[...].T + b1s[...]
    dt_o[...] = locsp[...] @ u3[...].T + b1t[...]


def _sf_body(al, ssrc, cs, ssta, ct, memp,
             fw2s, fb2s, fw2t, fb2t, pmw0, pmb0, pmw1, pmb1,
             mdw0a, mdw0b, mdb0, mdw1, mdb1,
             pw0, pb0, pw1, pb1, tw0, tb0, tw1, tb1, cw0, cb0, cw1, cb1,
             pred_o, predt_o, corr_o):
    a2s, a2t, apm, amd, apj, apt, apc = (al[0], al[1], al[2], al[3],
                                         al[4], al[5], al[6])
    invs = 1.0 / jnp.maximum(cs[...][0] + cs[...][1], 1.0)
    aggs = (ssrc[...][0] + ssrc[...][1]) * invs
    semb = _prelu(aggs @ fw2s[...].T + fb2s[...], a2s)
    mp = _prelu(memp[...] @ pmw0[...].T + pmb0[...], apm) @ pmw1[...].T + pmb1[...]
    mer = (_prelu(semb @ mdw0a[...].T + mp @ mdw0b[...].T + mdb0[...], amd)
           @ mdw1[...].T + mdb1[...])
    pred_o[...] = (_prelu(mer @ pw0[...].T + pb0[...], apj)
                   @ pw1[...].T + pb1[...]) * 5000.0
    predt_o[...] = (_prelu(mer @ tw0[...].T + tb0[...], apt)
                    @ tw1[...].T + tb1[...])
    invt = 1.0 / jnp.maximum(ct[...][0] + ct[...][1], 1.0)
    aggt = (ssta[...][0] + ssta[...][1]) * invt
    temb = _prelu(aggt @ fw2t[...].T + fb2t[...], a2t)
    corr_o[...] = (_prelu(temb @ cw0[...].T + cb0[...], apc)
                   @ cw1[...].T + cb1[...])


def _shape(n, f):
    return jax.ShapeDtypeStruct((n, f), _f32)


_S0 = pl.pallas_call(
    _s0_body,
    grid=(GRID,),
    in_specs=[_smem_spec(), _row_spec(24), _row_spec(24), _row_spec(FT),
              _full_spec((32, 24)), _full_spec((32, 8)), _full_spec((1, 32)),
              _full_spec((16, 32)), _full_spec((1, 16)),
              _full_spec((32, 24)), _full_spec((32, 8))],
    out_specs=[_row_spec(FT), _row_spec(F)],
    out_shape=[_shape(NPAD, FT), _shape(NPAD, F)],
)

_S1 = pl.pallas_call(
    _s1_body,
    grid=(GRID,),
    in_specs=[_smem_spec(), _row_spec(F), _row_spec(FT), _row_spec(4),
              _row_spec(4),
              _full_spec((F, F)), _full_spec((F, FT)), _full_spec((1, F)),
              _full_spec((F, F)), _full_spec((1, F)), _full_spec((F, 4))],
    out_specs=[_row_spec(F)] * 5,
    out_shape=[_shape(NPAD, F)] * 5,
)

_S2 = pl.pallas_call(
    _s2_body,
    grid=(GRID,),
    in_specs=[_smem_spec(), _row_spec(F), _row_spec(FT),
              _stack_spec(F), _stack_spec(F),
              _stack_spec(1), _stack_spec(1),
              _row_spec(F), _row_spec(F),
              _full_spec((64, F)), _full_spec((64, F)), _full_spec((64, F)),
              _full_spec((64, FT)), _full_spec((1, 64)),
              _full_spec((F, 64)), _full_spec((1, F)),
              _full_spec((F, 64)), _full_spec((1, F)),
              _full_spec((F, F)), _full_spec((1, F))],
    out_specs=[_row_spec(64), _row_spec(F), _row_spec(F)],
    out_shape=[_shape(NPAD, 64), _shape(NPAD, F), _shape(NPAD, F)],
)

_S3 = pl.pallas_call(
    _s3_body,
    grid=(GRID,),
    in_specs=[_smem_spec(), _row_spec(64), _row_spec(FT),
              _stack_spec(F), _stack_spec(F),
              _stack_spec(1), _stack_spec(1),
              _full_spec((F, 64)), _full_spec((F, F)), _full_spec((F, F)),
              _full_spec((F, FT)), _full_spec((1, F))],
    out_specs=[_row_spec(F)],
    out_shape=[_shape(NPAD, F)],
)

_SR1 = pl.pallas_call(
    _sr1_body,
    grid=(GRID,),
    in_specs=[_row_spec(F), _row_spec(FT), _row_spec(4), _row_spec(4),
              _full_spec((F, F)), _full_spec((F, FT)), _full_spec((F, 4)),
              _full_spec((F, F)), _full_spec((F, FT)), _full_spec((F, 4))],
    out_specs=[_row_spec(F), _row_spec(F)],
    out_shape=[_shape(NPAD, F), _shape(NPAD, F)],
)

_SR2 = pl.pallas_call(
    _sr2_body,
    grid=(GRID,),
    in_specs=[_smem_spec(), _row_spec(F),
              _full_spec((F, F)), _full_spec((1, F))],
    out_specs=[_row_spec(F)],
    out_shape=[_shape(NPAD, F)],
)

_SD = pl.pallas_call(
    _sd_body,
    grid=(1,),
    in_specs=[_full_spec((NSEG_SRC, 4)), _full_spec((NSEG_STA, 4)),
              _full_spec((F, 4)), _full_spec((1, F)),
              _full_spec((F, 4)), _full_spec((1, F))],
    out_specs=[_full_spec((NSEG_SRC, F)), _full_spec((NSEG_STA, F))],
    out_shape=[_shape(NSEG_SRC, F), _shape(NSEG_STA, F)],
)

_SF = pl.pallas_call(
    _sf_body,
    grid=(1,),
    in_specs=[_smem_spec(),
              _full_spec((NC, NSEG_SRC, F)), _full_spec((NC, NSEG_SRC, 1)),
              _full_spec((NC, NSEG_STA, F)), _full_spec((NC, NSEG_STA, 1)),
              _full_spec((NSEG_SRC, 8)),
              _full_spec((16, F)), _full_spec((1, 16)),
              _full_spec((16, F)), _full_spec((1, 16)),
              _full_spec((F, 8)), _full_spec((1, F)),
              _full_spec((16, F)), _full_spec((1, 16)),
              _full_spec((F, 16)), _full_spec((F, 16)), _full_spec((1, F)),
              _full_spec((F, F)), _full_spec((1, F)),
              _full_spec((F, F)), _full_spec((1, F)),
              _full_spec((8, F)), _full_spec((1, 8)),
              _full_spec((16, F)), _full_spec((1, 16)),
              _full_spec((8, 16)), _full_spec((1, 8)),
              _full_spec((16, 16)), _full_spec((1, 16)),
              _full_spec((8, 16)), _full_spec((1, 8))],
    out_specs=[_full_spec((NSEG_SRC, 8)), _full_spec((NSEG_SRC, 8)),
               _full_spec((NSEG_STA, 8))],
    out_shape=[_shape(NSEG_SRC, 8), _shape(NSEG_SRC, 8), _shape(NSEG_STA, 8)],
)


def _pad2(w, r, c):
    return jnp.zeros((r, c), _f32).at[:w.shape[0], :w.shape[1]].set(w)


def _padb(b, c):
    return jnp.zeros((1, c), _f32).at[0, :b.shape[0]].set(b)


def _pad_nodes(x, cols):
    return jnp.zeros((NPAD, cols), _f32).at[:x.shape[0], :x.shape[1]].set(x)


def _prep_edges(src, dst, nwin, pad_base, pad_mod):
    ne = src.shape[0]
    npad = NWRK * nwin * WSZ - ne
    srcp = jnp.concatenate([src, jnp.zeros((npad,), jnp.int32)])
    dstp = jnp.concatenate(
        [dst, pad_base + (jnp.arange(npad, dtype=jnp.int32) % pad_mod)])
    idx = jnp.stack([srcp.reshape(NWRK, nwin, WSZ),
                     dstp.reshape(NWRK, nwin, WSZ)], axis=2)
    return idx.reshape(NWRK * nwin, 2, WSZ)


def _prep_node_idx(idx):
    p = jnp.zeros((NPAD,), jnp.int32).at[:idx.shape[0]].set(idx)
    return p.reshape(NWRK * NW_R, WSZ)


def _avec(a):
    return jnp.full((16,), a, _f32)


def kernel(x, mask, A_in_pick, A_in_src, A_src_in_product, A_sta_in_product,
           A_src_in_sta, locs_cart, srcs_cart, memory, params):
    sta_id = A_src_in_sta[0]
    src_id = A_src_in_sta[1]

    t_src = jnp.zeros((NSEG_SRC, FT), _f32)
    t_src = t_src.at[:memory.shape[0], 0:4].set(memory)
    t_src = t_src.at[:srcs_cart.shape[0], 4:7].set(srcs_cart / 30000.0)
    t_sta = jnp.zeros((NSEG_STA, FT), _f32)
    t_sta = t_sta.at[:locs_cart.shape[0], 0:3].set(locs_cart / 30000.0)

    idx_src_nodes = _prep_node_idx(src_id)
    idx_sta_nodes = _prep_node_idx(sta_id)
    idx_pick = _prep_edges(A_in_pick[0], A_in_pick[1], NW_E, NREAL, 1024)
    idx_srce = _prep_edges(A_in_src[0], A_in_src[1], NW_E, NREAL, 1024)
    idx_rsrc = _prep_edges(A_src_in_product[0], A_src_in_product[1], NW_R,
                           1000, 24)
    idx_rsta = _prep_edges(A_sta_in_product[0], A_sta_in_product[1], NW_R,
                           500, 12)

    xp = _pad_nodes(x, 24)
    maskp = _pad_nodes(mask, 24)

    tg_src = _TAB_SRC(t_src, idx_src_nodes)
    tg_sta = _TAB_STA(t_sta, idx_sta_nodes)
    cnt_pick = _COUNTS_BIG(idx_pick).reshape(NC, NPAD, 1)
    cnt_srce = _COUNTS_BIG(idx_srce).reshape(NC, NPAD, 1)
    cnt_rsrc = _COUNTS_SRC(idx_rsrc).reshape(NC, NSEG_SRC, 1)
    cnt_rsta = _COUNTS_STA(idx_rsta).reshape(NC, NSEG_STA, 1)

    psta4 = tg_sta[:, 0:4]
    psrc4 = tg_src[:, 4:8]

    pe = params['embed_inpt']
    e0 = pe['layers'][0]['W']
    e1 = pe['layers'][1]['W']
    sx = jnp.zeros((F, 24), _f32).at[:18, :18].set(jnp.eye(18))
    sg = jnp.zeros((F, 8), _f32).at[18:22, 0:4].set(jnp.eye(4))
    al0 = jnp.array([pe['a']] + [0.0] * 7, _f32)
    mask_e, h = _S0(al0, xp, maskp, tg_src,
                    _pad2(e0[:, :18], 32, 24), _pad2(e0[:, 18:22], 32, 8),
                    _padb(pe['layers'][0]['b'], 32),
                    _pad2(e1, 16, 32), _padb(pe['layers'][1]['b'], 16),
                    sx, sg)

    for name in ['da1', 'da2', 'da3', 'da4', 'da5']:
        p = params[name]
        We = p['merge_edges']['W']
        wh = _pad2(We[:, :30], F, F)
        we3 = _pad2(We[:, 30:33], F, 4)
        bme = _padb(p['merge_edges']['b'], F)
        nin = 22 if name == 'da1' else 30
        wi = p['init_trns']['W']
        al1 = jnp.array([p['a_init'], p['a11'], p['a12'], 0, 0, 0, 0, 0], _f32)
        tr, a1, a2, g1, g2 = _S1(
            al1, h, mask_e, psta4, psrc4,
            _pad2(wi[:, :nin], F, F), _pad2(wi[:, nin:nin + 10], F, FT),
            _padb(p['init_trns']['b'], F), wh, bme, we3)

        am = _avec(p['a_merge'])
        p1 = _PROP(a1, g1, idx_pick, am)
        p2 = _PROP(a2, g2, idx_srce, am)

        w11, w12 = p['l1_t1_2']['W'], p['l1_t2_2']['W']
        wt = jnp.zeros((64, F), _f32)
        wt = wt.at[0:30, 0:30].set(w11[:, 0:30]).at[30:60, 0:30].set(w12[:, 0:30])
        wp1 = jnp.zeros((64, F), _f32).at[0:30, 0:30].set(w11[:, 30:60])
        wp2 = jnp.zeros((64, F), _f32).at[30:60, 0:30].set(w12[:, 30:60])
        wm = jnp.zeros((64, FT), _f32)
        wm = wm.at[0:30, 0:10].set(w11[:, 60:70]).at[30:60, 0:10].set(w12[:, 60:70])
        bb = jnp.zeros((1, 64), _f32)
        bb = bb.at[0, 0:30].set(p['l1_t1_2']['b']).at[0, 30:60].set(p['l1_t2_2']['b'])
        al2 = jnp.array([p['a1'], p['a21'], p['a22'], 0, 0, 0, 0, 0], _f32)
        trp, a3, a4 = _S2(
            al2, tr, mask_e, p1, p2, cnt_pick, cnt_srce, g1, g2,
            wt, wp1, wp2, wm, bb,
            _pad2(p['l2_t1_1']['W'], F, 64), _padb(p['l2_t1_1']['b'], F),
            _pad2(p['l2_t2_1']['W'], F, 64), _padb(p['l2_t2_1']['b'], F),
            wh, bme)

        p3 = _PROP(a3, g1, idx_pick, am)
        p4 = _PROP(a4, g2, idx_srce, am)

        w21, w22 = p['l2_t1_2']['W'], p['l2_t2_2']['W']
        wt2 = jnp.zeros((F, 64), _f32)
        wt2 = wt2.at[0:15, 0:60].set(w21[:, 0:60]).at[15:30, 0:60].set(w22[:, 0:60])
        wp3 = jnp.zeros((F, F), _f32).at[0:15, 0:30].set(w21[:, 60:90])
        wp4 = jnp.zeros((F, F), _f32).at[15:30, 0:30].set(w22[:, 60:90])
        wm2 = jnp.zeros((F, FT), _f32)
        wm2 = wm2.at[0:15, 0:10].set(w21[:, 90:100]).at[15:30, 0:10].set(w22[:, 90:100])
        bb2 = jnp.zeros((1, F), _f32)
        bb2 = bb2.at[0, 0:15].set(p['l2_t1_2']['b']).at[0, 15:30].set(p['l2_t2_2']['b'])
        al3 = jnp.array([p['a2'], 0, 0, 0, 0, 0, 0, 0], _f32)
        (h,) = _S3(al3, trp, mask_e, p3, p4, cnt_pick, cnt_srce,
                   wt2, wp3, wp4, wm2, bb2)

    pbs, pbt = params['bip_src'], params['bip_sta']
    v_w, u_w = pbs['fc1_0']['W'], pbt['fc1_0']['W']
    c_src, c_sta = _SR1(
        h, mask_e, psta4, psrc4,
        _pad2(v_w[:, :30], F, F), _pad2(v_w[:, 30:40], F, FT),
        _pad2(v_w[:, 40:43], F, 4),
        _pad2(u_w[:, :30], F, F), _pad2(u_w[:, 30:40], F, FT),
        _pad2(u_w[:, 40:43], F, 4))

    srcsp = jnp.zeros((NSEG_SRC, 4), _f32).at[:srcs_cart.shape[0], 0:3].set(
        srcs_cart / 30000.0)
    locsp = jnp.zeros((NSEG_STA, 4), _f32).at[:locs_cart.shape[0], 0:3].set(
        locs_cart / 30000.0)
    d_src, d_sta = _SD(srcsp, locsp,
                       _pad2(v_w[:, 40:43], F, 4), _padb(pbs['fc1_0']['b'], F),
                       _pad2(u_w[:, 40:43], F, 4), _padb(pbt['fc1_0']['b'], F))

    m1_src = _EMAP_SRC(c_src, d_src, idx_rsrc, _avec(pbs['a_fc1']))
    m1_sta = _EMAP_STA(c_sta, d_sta, idx_rsta, _avec(pbt['a_fc1']))

    als = jnp.array([pbs['a1'], 0, 0, 0, 0, 0, 0, 0], _f32)
    alt = jnp.array([pbt['a1'], 0, 0, 0, 0, 0, 0, 0], _f32)
    (m2_src,) = _SR2(als, m1_src, _pad2(pbs['fc1_2']['W'], F, F),
                     _padb(pbs['fc1_2']['b'], F))
    (m2_sta,) = _SR2(alt, m1_sta, _pad2(pbt['fc1_2']['W'], F, F),
                     _padb(pbt['fc1_2']['b'], F))

    s_src = _RSCAT_SRC(m2_src, idx_rsrc)
    s_sta = _RSCAT_STA(m2_sta, idx_rsta)

    pm, md = params['proj_memory'], params['merge_data']
    pj, pt, pc = params['proj'], params['proj_t'], params['proj_c']
    memp = jnp.zeros((NSEG_SRC, 8), _f32).at[:memory.shape[0], 0:4].set(memory)
    alf = jnp.array([pbs['a2'], pbt['a2'], pm['a'], md['a'],
                     pj['a'], pt['a'], pc['a'], 0.0], _f32)
    md0 = md['layers'][0]['W']
    pred, pred_t, corr = _SF(
        alf, s_src, cnt_rsrc, s_sta, cnt_rsta, memp,
        _pad2(pbs['fc2']['W'], 16, F), _padb(pbs['fc2']['b'], 16),
        _pad2(pbt['fc2']['W'], 16, F), _padb(pbt['fc2']['b'], 16),
        _pad2(pm['layers'][0]['W'], F, 8), _padb(pm['layers'][0]['b'], F),
        _pad2(pm['layers'][1]['W'], 16, F), _padb(pm['layers'][1]['b'], 16),
        _pad2(md0[:, :15], F, 16), _pad2(md0[:, 15:30], F, 16),
        _padb(md['layers'][0]['b'], F),
        _pad2(md['layers'][1]['W'], F, F), _padb(md['layers'][1]['b'], F),
        _pad2(pj['layers'][0]['W'], F, F), _padb(pj['layers'][0]['b'], F),
        _pad2(pj['layers'][1]['W'], 8, F), _padb(pj['layers'][1]['b'], 8),
        _pad2(pt['layers'][0]['W'], 16, F), _padb(pt['layers'][0]['b'], 16),
        _pad2(pt['layers'][1]['W'], 8, 16), _padb(pt['layers'][1]['b'], 8),
        _pad2(pc['layers'][0]['W'], 16, 16), _padb(pc['layers'][0]['b'], 16),
        _pad2(pc['layers'][1]['W'], 8, 16), _padb(pc['layers'][1]['b'], 8))

    return (pred[:1000, :3], pred_t[:1000, :1], corr[:500, :2])


# speedup vs baseline: 8.7911x; 1.0453x over previous
import functools

import jax
import jax.numpy as jnp
from jax import lax
from jax.experimental import pallas as pl
from jax.experimental.pallas import tpu as pltpu
from jax.experimental.pallas import tpu_sc as plsc

NREAL = 50000
NPAD = 53248
F = 32
FT = 16
NE = 1600000
WSZ = 128
NWRK = 32
NC = 2
NSUB = 16
NW_E = 392
EPAD = NWRK * NW_E * WSZ
NW_R = 13
NSEG_SRC = 1024
NSEG_STA = 512
BM = 512
GRID = NPAD // BM

_f32 = jnp.float32


def _mesh():
    return plsc.VectorSubcoreMesh(core_axis_name="c", subcore_axis_name="s")


def _make_prop():
    @functools.partial(
        pl.kernel,
        compiler_params=pltpu.CompilerParams(use_tc_tiling_on_sc=False),
        out_type=jax.ShapeDtypeStruct((NC, NPAD, F), _f32),
        mesh=_mesh(),
        scratch_types=(
            [pltpu.VMEM((2, WSZ), jnp.int32) for _ in range(2)]
            + [pltpu.VMEM((1, WSZ), jnp.int32) for _ in range(2)]
            + [pltpu.VMEM((WSZ, F), _f32) for _ in range(2)]
            + [pltpu.VMEM((WSZ, F), _f32) for _ in range(2)]
            + [pltpu.VMEM((16,), _f32),
               pltpu.VMEM((64, F), _f32),
               pltpu.VMEM_SHARED((NPAD, F), _f32)]
            + [pltpu.SemaphoreType.DMA for _ in range(6)]
        ),
    )
    def prop(a_hbm, g_hbm, idx_hbm, alpha_hbm, out_hbm,
             ib0, ib1, sb0, sb1, a0, a1, g0, g1, alphav, zbuf, acc,
             si0, si1, sg0, sg1, ss0, ss1):
        ibuf = (ib0, ib1)
        sibuf = (sb0, sb1)
        abuf = (a0, a1)
        gbuf = (g0, g1)
        si = (si0, si1)
        sg = (sg0, sg1)
        ss = (ss0, ss1)

        c = lax.axis_index("c")
        s = lax.axis_index("s")
        wid = s * NC + c
        rows_per_sub = NPAD // NSUB
        rows0 = s * rows_per_sub

        def zb(i, _):
            zbuf[i, pl.ds(0, 16)] = jnp.zeros((16,), _f32)
            zbuf[i, pl.ds(16, 16)] = jnp.zeros((16,), _f32)
            return 0
        lax.fori_loop(0, 64, zb, 0)

        def zc(k, _):
            pltpu.sync_copy(zbuf, acc.at[pl.ds(rows0 + k * 64, 64)])
            return 0
        lax.fori_loop(0, rows_per_sub // 64, zc, 0)

        pltpu.sync_copy(alpha_hbm, alphav)
        av = alphav[...]
        plsc.subcore_barrier()

        wbase = wid * NW_E

        def fire_idx(k, w):
            pltpu.async_copy(idx_hbm.at[wbase + w], ibuf[k], si[k])

        def wait_idx(k, w):
            pltpu.make_async_copy(idx_hbm.at[wbase + w], ibuf[k],
                                  si[k]).wait()

        def fire_gather(k):
            pltpu.async_copy(a_hbm.at[ibuf[k].at[0]], abuf[k], sg[k])
            pltpu.async_copy(g_hbm.at[ibuf[k].at[1]], gbuf[k], sg[k])

        def wait_gather(k):
            pltpu.make_async_copy(a_hbm.at[ibuf[k].at[0]], abuf[k],
                                  sg[k]).wait()
            pltpu.make_async_copy(g_hbm.at[ibuf[k].at[1]], gbuf[k],
                                  sg[k]).wait()

        def copy_scatter_idx(k):
            def cpb(i, _):
                sibuf[k][0, pl.ds(i * 16, 16)] = ibuf[k][1, pl.ds(i * 16, 16)]
                return 0
            lax.fori_loop(0, WSZ // 16, cpb, 0, unroll=8)

        def fire_scat(k):
            pltpu.async_copy(abuf[k], acc.at[sibuf[k].at[0]], ss[k],
                             add=True)

        def wait_scat(k):
            pltpu.make_async_copy(abuf[k], acc.at[sibuf[k].at[0]],
                                  ss[k]).wait()

        def compute(k):
            ab, gb = abuf[k], gbuf[k]

            def cb(i, _):
                for hh in (0, 16):
                    d = ab[i, pl.ds(hh, 16)] - gb[i, pl.ds(hh, 16)]
                    ab[i, pl.ds(hh, 16)] = (jnp.maximum(d, 0.0)
                                            + av * jnp.minimum(d, 0.0))
                return 0
            lax.fori_loop(0, WSZ, cb, 0, unroll=8)

        fire_idx(0, 0)
        fire_idx(1, 1)

        def body(j, _):
            w0 = 2 * j
            for k in range(2):
                wait_idx(k, w0 + k)

                @pl.when(j > 0)
                def _():
                    wait_scat(k)
                copy_scatter_idx(k)
                fire_gather(k)
            for k in range(2):
                wait_gather(k)

                @pl.when(w0 + k + 2 < NW_E)
                def _():
                    fire_idx(k, w0 + k + 2)
                compute(k)
                fire_scat(k)
            return 0
        lax.fori_loop(0, NW_E // 2, body, 0)
        wait_scat(0)
        wait_scat(1)

        plsc.subcore_barrier()
        pltpu.sync_copy(acc.at[pl.ds(rows0, rows_per_sub)],
                        out_hbm.at[c, pl.ds(rows0, rows_per_sub)])

    return prop


def _make_counts_big():
    rows_per_sub = NPAD // NSUB
    nb = 4

    @functools.partial(
        pl.kernel,
        compiler_params=pltpu.CompilerParams(use_tc_tiling_on_sc=False),
        out_type=jax.ShapeDtypeStruct((NC, NPAD), _f32),
        mesh=_mesh(),
        scratch_types=(
            [pltpu.VMEM((2, WSZ), jnp.int32) for _ in range(4)]
            + [pltpu.VMEM((1, WSZ), jnp.int32) for _ in range(4)]
            + [pltpu.VMEM((WSZ,), _f32), pltpu.VMEM((WSZ,), _f32)]
            + [pltpu.SemaphoreType.DMA for _ in range(8)]
            + [pltpu.VMEM_SHARED((NPAD,), _f32)]
        ),
    )
    def counts(idx_hbm, out_hbm, *scr):
        ibuf = scr[0:4]
        sibuf = scr[4:8]
        ones, zv = scr[8:10]
        si = scr[10:14]
        ss = scr[14:18]
        acc = scr[18]

        c = lax.axis_index("c")
        s = lax.axis_index("s")
        wid = s * NC + c
        rows0 = s * rows_per_sub

        def ib(i, _):
            ones[pl.ds(i * 16, 16)] = jnp.full((16,), 1.0, _f32)
            zv[pl.ds(i * 16, 16)] = jnp.zeros((16,), _f32)
            return 0
        lax.fori_loop(0, WSZ // 16, ib, 0)

        def zc(k, _):
            pltpu.sync_copy(zv, acc.at[pl.ds(rows0 + k * WSZ, WSZ)])
            return 0
        lax.fori_loop(0, rows_per_sub // WSZ, zc, 0)
        plsc.subcore_barrier()

        wbase = wid * NW_E

        def fire_idx(k, w):
            pltpu.async_copy(idx_hbm.at[wbase + w], ibuf[k], si[k])

        def wait_idx(k, w):
            pltpu.make_async_copy(idx_hbm.at[wbase + w], ibuf[k],
                                  si[k]).wait()

        def copy_scatter_idx(k):
            def cpb(i, _):
                sibuf[k][0, pl.ds(i * 16, 16)] = ibuf[k][1, pl.ds(i * 16, 16)]
                return 0
            lax.fori_loop(0, WSZ // 16, cpb, 0, unroll=8)

        def fire_scat(k):
            pltpu.async_copy(ones, acc.at[sibuf[k].at[0]], ss[k], add=True)

        def wait_scat(k):
            pltpu.make_async_copy(ones, acc.at[sibuf[k].at[0]],
                                  ss[k]).wait()

        for k in range(nb):
            fire_idx(k, k)

        def body(j, _):
            w0 = nb * j
            for k in range(nb):
                wait_idx(k, w0 + k)

                @pl.when(j > 0)
                def _():
                    wait_scat(k)
                copy_scatter_idx(k)

                @pl.when(w0 + k + nb < NW_E)
                def _():
                    fire_idx(k, w0 + k + nb)
                fire_scat(k)
            return 0
        lax.fori_loop(0, NW_E // nb, body, 0)
        for k in range(nb):
            wait_scat(k)

        plsc.subcore_barrier()
        pltpu.sync_copy(acc.at[pl.ds(rows0, rows_per_sub)],
                        out_hbm.at[c, pl.ds(rows0, rows_per_sub)])

    return counts


def _make_counts(nseg, nwin):
    rows_per_sub = nseg // NSUB

    @functools.partial(
        pl.kernel,
        compiler_params=pltpu.CompilerParams(use_tc_tiling_on_sc=False),
        out_type=jax.ShapeDtypeStruct((NC, nseg), _f32),
        mesh=_mesh(),
        scratch_types=[
            pltpu.VMEM((2, WSZ), jnp.int32),
            pltpu.VMEM((WSZ,), _f32),
            pltpu.VMEM_SHARED((nseg,), _f32),
        ],
    )
    def counts(idx_hbm, out_hbm, idxb, ones, acc):
        c = lax.axis_index("c")
        s = lax.axis_index("s")
        wid = s * NC + c
        rows0 = s * rows_per_sub

        def zb(i, _):
            ones[pl.ds(i * 16, 16)] = jnp.zeros((16,), _f32)
            return 0
        lax.fori_loop(0, WSZ // 16, zb, 0)
        if rows_per_sub <= WSZ:
            pltpu.sync_copy(ones.at[pl.ds(0, rows_per_sub)],
                            acc.at[pl.ds(rows0, rows_per_sub)])
        else:
            def zc(k, _):
                pltpu.sync_copy(ones, acc.at[pl.ds(rows0 + k * WSZ, WSZ)])
                return 0
            lax.fori_loop(0, rows_per_sub // WSZ, zc, 0)

        def ob(i, _):
            ones[pl.ds(i * 16, 16)] = jnp.full((16,), 1.0, _f32)
            return 0
        lax.fori_loop(0, WSZ // 16, ob, 0)
        plsc.subcore_barrier()

        wbase = wid * nwin

        def body(w, _):
            pltpu.sync_copy(idx_hbm.at[wbase + w], idxb)
            pltpu.sync_copy(ones, acc.at[idxb.at[1]], add=True)
            return 0
        lax.fori_loop(0, nwin, body, 0)

        plsc.subcore_barrier()
        pltpu.sync_copy(acc.at[pl.ds(rows0, rows_per_sub)],
                        out_hbm.at[c, pl.ds(rows0, rows_per_sub)])

    return counts


def _make_tab(ntab):
    @functools.partial(
        pl.kernel,
        compiler_params=pltpu.CompilerParams(use_tc_tiling_on_sc=False),
        out_type=jax.ShapeDtypeStruct((NPAD, FT), _f32),
        mesh=_mesh(),
        scratch_types=[
            pltpu.VMEM((WSZ,), jnp.int32),
            pltpu.VMEM((WSZ, FT), _f32),
            pltpu.SemaphoreType.DMA,
        ],
    )
    def tab(t_hbm, idx_hbm, out_hbm, idxb, tbuf, sem):
        c = lax.axis_index("c")
        s = lax.axis_index("s")
        wid = s * NC + c
        ebase = wid * NW_R * WSZ

        def body(w, _):
            pltpu.sync_copy(idx_hbm.at[wid * NW_R + w], idxb)
            pltpu.async_copy(t_hbm.at[idxb], tbuf, sem).wait()
            pltpu.sync_copy(tbuf, out_hbm.at[pl.ds(ebase + w * WSZ, WSZ)])
            return 0
        lax.fori_loop(0, NW_R, body, 0)

    return tab


def _make_emap(nseg):
    @functools.partial(
        pl.kernel,
        compiler_params=pltpu.CompilerParams(use_tc_tiling_on_sc=False),
        out_type=jax.ShapeDtypeStruct((NPAD, F), _f32),
        mesh=_mesh(),
        scratch_types=[
            pltpu.VMEM((2, WSZ), jnp.int32),
            pltpu.VMEM((WSZ, F), _f32),
            pltpu.VMEM((WSZ, F), _f32),
            pltpu.VMEM((16,), _f32),
            pltpu.SemaphoreType.DMA,
            pltpu.SemaphoreType.DMA,
        ],
    )
    def emap(c_hbm, d_hbm, idx_hbm, alpha_hbm, out_hbm,
             idxb, cbuf, dbuf, alphav, sc_, sd_):
        c = lax.axis_index("c")
        s = lax.axis_index("s")
        wid = s * NC + c
        ebase = wid * NW_R * WSZ
        pltpu.sync_copy(alpha_hbm, alphav)
        av = alphav[...]

        def body(w, _):
            pltpu.sync_copy(idx_hbm.at[wid * NW_R + w], idxb)
            pltpu.async_copy(c_hbm.at[idxb.at[0]], cbuf, sc_)
            pltpu.async_copy(d_hbm.at[idxb.at[1]], dbuf, sd_)
            pltpu.make_async_copy(c_hbm.at[idxb.at[0]], cbuf, sc_).wait()
            pltpu.make_async_copy(d_hbm.at[idxb.at[1]], dbuf, sd_).wait()

            def cb(i, _):
                for hh in (0, 16):
                    d = cbuf[i, pl.ds(hh, 16)] + dbuf[i, pl.ds(hh, 16)]
                    cbuf[i, pl.ds(hh, 16)] = (jnp.maximum(d, 0.0)
                                              + av * jnp.minimum(d, 0.0))
                return 0
            lax.fori_loop(0, WSZ, cb, 0, unroll=4)
            pltpu.sync_copy(cbuf, out_hbm.at[pl.ds(ebase + w * WSZ, WSZ)])
            return 0
        lax.fori_loop(0, NW_R, body, 0)

    return emap


def _make_rscatter(nseg):
    rows_per_sub = nseg // NSUB

    @functools.partial(
        pl.kernel,
        compiler_params=pltpu.CompilerParams(use_tc_tiling_on_sc=False),
        out_type=jax.ShapeDtypeStruct((NC, nseg, F), _f32),
        mesh=_mesh(),
        scratch_types=[
            pltpu.VMEM((2, WSZ), jnp.int32),
            pltpu.VMEM((WSZ, F), _f32),
            pltpu.VMEM((WSZ, F), _f32),
            pltpu.VMEM_SHARED((nseg, F), _f32),
            pltpu.SemaphoreType.DMA,
        ],
    )
    def rscatter(m_hbm, idx_hbm, out_hbm, idxb, mbuf, zbuf, acc, sem):
        c = lax.axis_index("c")
        s = lax.axis_index("s")
        wid = s * NC + c
        rows0 = s * rows_per_sub
        ebase = wid * NW_R * WSZ

        def zb(i, _):
            zbuf[i, pl.ds(0, 16)] = jnp.zeros((16,), _f32)
            zbuf[i, pl.ds(16, 16)] = jnp.zeros((16,), _f32)
            return 0
        lax.fori_loop(0, WSZ, zb, 0)
        pltpu.sync_copy(zbuf.at[pl.ds(0, rows_per_sub)],
                        acc.at[pl.ds(rows0, rows_per_sub)])
        plsc.subcore_barrier()

        def body(w, _):
            pltpu.sync_copy(idx_hbm.at[wid * NW_R + w], idxb)
            pltpu.async_copy(m_hbm.at[pl.ds(ebase + w * WSZ, WSZ)],
                             mbuf, sem).wait()
            pltpu.sync_copy(mbuf, acc.at[idxb.at[1]], add=True)
            return 0
        lax.fori_loop(0, NW_R, body, 0)

        plsc.subcore_barrier()
        pltpu.sync_copy(acc.at[pl.ds(rows0, rows_per_sub)],
                        out_hbm.at[c, pl.ds(rows0, rows_per_sub)])

    return rscatter


_PROP = _make_prop()
_COUNTS_BIG = _make_counts_big()
_COUNTS_SRC = _make_counts(NSEG_SRC, NW_R)
_COUNTS_STA = _make_counts(NSEG_STA, NW_R)
_TAB_SRC = _make_tab(NSEG_SRC)
_TAB_STA = _make_tab(NSEG_STA)
_EMAP_SRC = _make_emap(NSEG_SRC)
_EMAP_STA = _make_emap(NSEG_STA)
_RSCAT_SRC = _make_rscatter(NSEG_SRC)
_RSCAT_STA = _make_rscatter(NSEG_STA)


def _prelu(x, a):
    return jnp.where(x >= 0, x, a * x)


def _row_spec(f):
    return pl.BlockSpec((BM, f), lambda i: (i, 0))


def _stack_spec(f):
    return pl.BlockSpec((2, BM, f), lambda i: (0, i, 0))


def _full_spec(shape):
    nd = len(shape)
    return pl.BlockSpec(shape, lambda i: (0,) * nd)


def _smem_spec():
    return pl.BlockSpec(memory_space=pltpu.SMEM)


def _s0_body(al, xp, mk, tg, e0m, e0g, b0, e1, b1, sx, sg, me_o, h0_o):
    a_e = al[0]
    t8 = tg[...][:, 0:8]
    h = _prelu(mk[...] @ e0m[...].T + t8 @ e0g[...].T + b0[...], a_e)
    me_o[...] = h @ e1[...].T + b1[...]
    h0_o[...] = xp[...] @ sx[...].T + t8 @ sg[...].T


def _s1_body(al, h, me, psta, psrc, wih, wim, bi, wh, bme, we3,
             tr_o, a1_o, a2_o, g1_o, g2_o):
    a_init, a11, a12 = al[0], al[1], al[2]
    tr = _prelu(h[...] @ wih[...].T + me[...] @ wim[...].T + bi[...], a_init)
    g1 = psta[...] @ we3[...].T
    g2 = psrc[...] @ we3[...].T
    tr_o[...] = tr
    a1_o[...] = _prelu(tr, a11) @ wh[...].T + bme[...] + g1
    a2_o[...] = _prelu(tr, a12) @ wh[...].T + bme[...] + g2
    g1_o[...] = g1
    g2_o[...] = g2


def _s2_body(al, tr, me, p1, p2, c1, c2, g1, g2,
             wt, wp1, wp2, wm, bb, w21, b21, w22, b22, wh, bme,
             trp_o, a3_o, a4_o):
    a1s, a21, a22 = al[0], al[1], al[2]
    ic1 = 1.0 / jnp.maximum(c1[...][0] + c1[...][1], 1.0)
    ic2 = 1.0 / jnp.maximum(c2[...][0] + c2[...][1], 1.0)
    prop1 = (p1[...][0] + p1[...][1]) * ic1
    prop2 = (p2[...][0] + p2[...][1]) * ic2
    trp = _prelu(tr[...] @ wt[...].T + prop1 @ wp1[...].T
                 + prop2 @ wp2[...].T + me[...] @ wm[...].T + bb[...], a1s)
    u1 = _prelu(trp @ w21[...].T + b21[...], a21)
    u2 = _prelu(trp @ w22[...].T + b22[...], a22)
    trp_o[...] = trp
    a3_o[...] = u1 @ wh[...].T + bme[...] + g1[...]
    a4_o[...] = u2 @ wh[...].T + bme[...] + g2[...]


def _s3_body(al, trp, me, p3, p4, c1, c2, wt, wp1, wp2, wm, bb, h_o):
    a2s = al[0]
    ic1 = 1.0 / jnp.maximum(c1[...][0] + c1[...][1], 1.0)
    ic2 = 1.0 / jnp.maximum(c2[...][0] + c2[...][1], 1.0)
    prop3 = (p3[...][0] + p3[...][1]) * ic1
    prop4 = (p4[...][0] + p4[...][1]) * ic2
    h_o[...] = _prelu(trp[...] @ wt[...].T + prop3 @ wp1[...].T
                      + prop4 @ wp2[...].T + me[...] @ wm[...].T + bb[...], a2s)


def _sr1_body(h, me, psta, psrc, v1, v2, v3, u1, u2, u3, cs_o, ct_o):
    cs_o[...] = h[...] @ v1[...].T + me[...] @ v2[...].T - psta[...] @ v3[...].T
    ct_o[...] = h[...] @ u1[...].T + me[...] @ u2[...].T - psrc[...] @ u3[...].T


def _sr2_body(al, m1, w, b, o):
    o[...] = _prelu(m1[...] @ w[...].T + b[...], al[0])


def _sd_body(srcsp, locsp, v3, b1s, u3, b1t, ds_o, dt_o):
    ds_o[...] = srcsp[...] @ v3[...].T + b1s[...]
    dt_o[...] = locsp[...] @ u3[...].T + b1t[...]


def _sf_body(al, ssrc, cs, ssta, ct, memp,
             fw2s, fb2s, fw2t, fb2t, pmw0, pmb0, pmw1, pmb1,
             mdw0a, mdw0b, mdb0, mdw1, mdb1,
             pw0, pb0, pw1, pb1, tw0, tb0, tw1, tb1, cw0, cb0, cw1, cb1,
             pred_o, predt_o, corr_o):
    a2s, a2t, apm, amd, apj, apt, apc = (al[0], al[1], al[2], al[3],
                                         al[4], al[5], al[6])
    invs = 1.0 / jnp.maximum(cs[...][0] + cs[...][1], 1.0)
    aggs = (ssrc[...][0] + ssrc[...][1]) * invs
    semb = _prelu(aggs @ fw2s[...].T + fb2s[...], a2s)
    mp = _prelu(memp[...] @ pmw0[...].T + pmb0[...], apm) @ pmw1[...].T + pmb1[...]
    mer = (_prelu(semb @ mdw0a[...].T + mp @ mdw0b[...].T + mdb0[...], amd)
           @ mdw1[...].T + mdb1[...])
    pred_o[...] = (_prelu(mer @ pw0[...].T + pb0[...], apj)
                   @ pw1[...].T + pb1[...]) * 5000.0
    predt_o[...] = (_prelu(mer @ tw0[...].T + tb0[...], apt)
                    @ tw1[...].T + tb1[...])
    invt = 1.0 / jnp.maximum(ct[...][0] + ct[...][1], 1.0)
    aggt = (ssta[...][0] + ssta[...][1]) * invt
    temb = _prelu(aggt @ fw2t[...].T + fb2t[...], a2t)
    corr_o[...] = (_prelu(temb @ cw0[...].T + cb0[...], apc)
                   @ cw1[...].T + cb1[...])


def _shape(n, f):
    return jax.ShapeDtypeStruct((n, f), _f32)


_S0 = pl.pallas_call(
    _s0_body,
    grid=(GRID,),
    in_specs=[_smem_spec(), _row_spec(24), _row_spec(24), _row_spec(FT),
              _full_spec((32, 24)), _full_spec((32, 8)), _full_spec((1, 32)),
              _full_spec((16, 32)), _full_spec((1, 16)),
              _full_spec((32, 24)), _full_spec((32, 8))],
    out_specs=[_row_spec(FT), _row_spec(F)],
    out_shape=[_shape(NPAD, FT), _shape(NPAD, F)],
)

_S1 = pl.pallas_call(
    _s1_body,
    grid=(GRID,),
    in_specs=[_smem_spec(), _row_spec(F), _row_spec(FT), _row_spec(4),
              _row_spec(4),
              _full_spec((F, F)), _full_spec((F, FT)), _full_spec((1, F)),
              _full_spec((F, F)), _full_spec((1, F)), _full_spec((F, 4))],
    out_specs=[_row_spec(F)] * 5,
    out_shape=[_shape(NPAD, F)] * 5,
)

_S2 = pl.pallas_call(
    _s2_body,
    grid=(GRID,),
    in_specs=[_smem_spec(), _row_spec(F), _row_spec(FT),
              _stack_spec(F), _stack_spec(F),
              _stack_spec(1), _stack_spec(1),
              _row_spec(F), _row_spec(F),
              _full_spec((64, F)), _full_spec((64, F)), _full_spec((64, F)),
              _full_spec((64, FT)), _full_spec((1, 64)),
              _full_spec((F, 64)), _full_spec((1, F)),
              _full_spec((F, 64)), _full_spec((1, F)),
              _full_spec((F, F)), _full_spec((1, F))],
    out_specs=[_row_spec(64), _row_spec(F), _row_spec(F)],
    out_shape=[_shape(NPAD, 64), _shape(NPAD, F), _shape(NPAD, F)],
)

_S3 = pl.pallas_call(
    _s3_body,
    grid=(GRID,),
    in_specs=[_smem_spec(), _row_spec(64), _row_spec(FT),
              _stack_spec(F), _stack_spec(F),
              _stack_spec(1), _stack_spec(1),
              _full_spec((F, 64)), _full_spec((F, F)), _full_spec((F, F)),
              _full_spec((F, FT)), _full_spec((1, F))],
    out_specs=[_row_spec(F)],
    out_shape=[_shape(NPAD, F)],
)

_SR1 = pl.pallas_call(
    _sr1_body,
    grid=(GRID,),
    in_specs=[_row_spec(F), _row_spec(FT), _row_spec(4), _row_spec(4),
              _full_spec((F, F)), _full_spec((F, FT)), _full_spec((F, 4)),
              _full_spec((F, F)), _full_spec((F, FT)), _full_spec((F, 4))],
    out_specs=[_row_spec(F), _row_spec(F)],
    out_shape=[_shape(NPAD, F), _shape(NPAD, F)],
)

_SR2 = pl.pallas_call(
    _sr2_body,
    grid=(GRID,),
    in_specs=[_smem_spec(), _row_spec(F),
              _full_spec((F, F)), _full_spec((1, F))],
    out_specs=[_row_spec(F)],
    out_shape=[_shape(NPAD, F)],
)

_SD = pl.pallas_call(
    _sd_body,
    grid=(1,),
    in_specs=[_full_spec((NSEG_SRC, 4)), _full_spec((NSEG_STA, 4)),
              _full_spec((F, 4)), _full_spec((1, F)),
              _full_spec((F, 4)), _full_spec((1, F))],
    out_specs=[_full_spec((NSEG_SRC, F)), _full_spec((NSEG_STA, F))],
    out_shape=[_shape(NSEG_SRC, F), _shape(NSEG_STA, F)],
)

_SF = pl.pallas_call(
    _sf_body,
    grid=(1,),
    in_specs=[_smem_spec(),
              _full_spec((NC, NSEG_SRC, F)), _full_spec((NC, NSEG_SRC, 1)),
              _full_spec((NC, NSEG_STA, F)), _full_spec((NC, NSEG_STA, 1)),
              _full_spec((NSEG_SRC, 8)),
              _full_spec((16, F)), _full_spec((1, 16)),
              _full_spec((16, F)), _full_spec((1, 16)),
              _full_spec((F, 8)), _full_spec((1, F)),
              _full_spec((16, F)), _full_spec((1, 16)),
              _full_spec((F, 16)), _full_spec((F, 16)), _full_spec((1, F)),
              _full_spec((F, F)), _full_spec((1, F)),
              _full_spec((F, F)), _full_spec((1, F)),
              _full_spec((8, F)), _full_spec((1, 8)),
              _full_spec((16, F)), _full_spec((1, 16)),
              _full_spec((8, 16)), _full_spec((1, 8)),
              _full_spec((16, 16)), _full_spec((1, 16)),
              _full_spec((8, 16)), _full_spec((1, 8))],
    out_specs=[_full_spec((NSEG_SRC, 8)), _full_spec((NSEG_SRC, 8)),
               _full_spec((NSEG_STA, 8))],
    out_shape=[_shape(NSEG_SRC, 8), _shape(NSEG_SRC, 8), _shape(NSEG_STA, 8)],
)


def _pad2(w, r, c):
    return jnp.zeros((r, c), _f32).at[:w.shape[0], :w.shape[1]].set(w)


def _padb(b, c):
    return jnp.zeros((1, c), _f32).at[0, :b.shape[0]].set(b)


def _pad_nodes(x, cols):
    return jnp.zeros((NPAD, cols), _f32).at[:x.shape[0], :x.shape[1]].set(x)


def _prep_edges(src, dst, nwin, pad_base, pad_mod):
    ne = src.shape[0]
    npad = NWRK * nwin * WSZ - ne
    srcp = jnp.concatenate([src, jnp.zeros((npad,), jnp.int32)])
    dstp = jnp.concatenate(
        [dst, pad_base + (jnp.arange(npad, dtype=jnp.int32) % pad_mod)])
    idx = jnp.stack([srcp.reshape(NWRK, nwin, WSZ),
                     dstp.reshape(NWRK, nwin, WSZ)], axis=2)
    return idx.reshape(NWRK * nwin, 2, WSZ)


def _prep_edges_big(src, dst, pad_base, pad_mod):
    ne = src.shape[0]
    npad = EPAD - ne
    srcp = jnp.concatenate([src, jnp.zeros((npad,), jnp.int32)])
    dstp = jnp.concatenate(
        [dst, pad_base + (jnp.arange(npad, dtype=jnp.int32) % pad_mod)])
    idx = jnp.stack([srcp.reshape(NWRK, NW_E, WSZ),
                     dstp.reshape(NWRK, NW_E, WSZ)], axis=2)
    return idx.reshape(NWRK * NW_E, 2, WSZ)


def _prep_node_idx(idx):
    p = jnp.zeros((NPAD,), jnp.int32).at[:idx.shape[0]].set(idx)
    return p.reshape(NWRK * NW_R, WSZ)


def _avec(a):
    return jnp.full((16,), a, _f32)


def kernel(x, mask, A_in_pick, A_in_src, A_src_in_product, A_sta_in_product,
           A_src_in_sta, locs_cart, srcs_cart, memory, params):
    sta_id = A_src_in_sta[0]
    src_id = A_src_in_sta[1]

    t_src = jnp.zeros((NSEG_SRC, FT), _f32)
    t_src = t_src.at[:memory.shape[0], 0:4].set(memory)
    t_src = t_src.at[:srcs_cart.shape[0], 4:7].set(srcs_cart / 30000.0)
    t_sta = jnp.zeros((NSEG_STA, FT), _f32)
    t_sta = t_sta.at[:locs_cart.shape[0], 0:3].set(locs_cart / 30000.0)

    idx_src_nodes = _prep_node_idx(src_id)
    idx_sta_nodes = _prep_node_idx(sta_id)
    idx_pick = _prep_edges_big(A_in_pick[0], A_in_pick[1], NREAL, 1024)
    idx_srce = _prep_edges_big(A_in_src[0], A_in_src[1], NREAL, 1024)
    idx_rsrc = _prep_edges(A_src_in_product[0], A_src_in_product[1], NW_R,
                           1000, 24)
    idx_rsta = _prep_edges(A_sta_in_product[0], A_sta_in_product[1], NW_R,
                           500, 12)

    xp = _pad_nodes(x, 24)
    maskp = _pad_nodes(mask, 24)

    tg_src = _TAB_SRC(t_src, idx_src_nodes)
    tg_sta = _TAB_STA(t_sta, idx_sta_nodes)
    cnt_pick = _COUNTS_BIG(idx_pick).reshape(NC, NPAD, 1)
    cnt_srce = _COUNTS_BIG(idx_srce).reshape(NC, NPAD, 1)
    cnt_rsrc = _COUNTS_SRC(idx_rsrc).reshape(NC, NSEG_SRC, 1)
    cnt_rsta = _COUNTS_STA(idx_rsta).reshape(NC, NSEG_STA, 1)

    psta4 = tg_sta[:, 0:4]
    psrc4 = tg_src[:, 4:8]

    pe = params['embed_inpt']
    e0 = pe['layers'][0]['W']
    e1 = pe['layers'][1]['W']
    sx = jnp.zeros((F, 24), _f32).at[:18, :18].set(jnp.eye(18))
    sg = jnp.zeros((F, 8), _f32).at[18:22, 0:4].set(jnp.eye(4))
    al0 = jnp.array([pe['a']] + [0.0] * 7, _f32)
    mask_e, h = _S0(al0, xp, maskp, tg_src,
                    _pad2(e0[:, :18], 32, 24), _pad2(e0[:, 18:22], 32, 8),
                    _padb(pe['layers'][0]['b'], 32),
                    _pad2(e1, 16, 32), _padb(pe['layers'][1]['b'], 16),
                    sx, sg)

    for name in ['da1', 'da2', 'da3', 'da4', 'da5']:
        p = params[name]
        We = p['merge_edges']['W']
        wh = _pad2(We[:, :30], F, F)
        we3 = _pad2(We[:, 30:33], F, 4)
        bme = _padb(p['merge_edges']['b'], F)
        nin = 22 if name == 'da1' else 30
        wi = p['init_trns']['W']
        al1 = jnp.array([p['a_init'], p['a11'], p['a12'], 0, 0, 0, 0, 0], _f32)
        tr, a1, a2, g1, g2 = _S1(
            al1, h, mask_e, psta4, psrc4,
            _pad2(wi[:, :nin], F, F), _pad2(wi[:, nin:nin + 10], F, FT),
            _padb(p['init_trns']['b'], F), wh, bme, we3)

        am = _avec(p['a_merge'])
        p1 = _PROP(a1, g1, idx_pick, am)
        p2 = _PROP(a2, g2, idx_srce, am)

        w11, w12 = p['l1_t1_2']['W'], p['l1_t2_2']['W']
        wt = jnp.zeros((64, F), _f32)
        wt = wt.at[0:30, 0:30].set(w11[:, 0:30]).at[30:60, 0:30].set(w12[:, 0:30])
        wp1 = jnp.zeros((64, F), _f32).at[0:30, 0:30].set(w11[:, 30:60])
        wp2 = jnp.zeros((64, F), _f32).at[30:60, 0:30].set(w12[:, 30:60])
        wm = jnp.zeros((64, FT), _f32)
        wm = wm.at[0:30, 0:10].set(w11[:, 60:70]).at[30:60, 0:10].set(w12[:, 60:70])
        bb = jnp.zeros((1, 64), _f32)
        bb = bb.at[0, 0:30].set(p['l1_t1_2']['b']).at[0, 30:60].set(p['l1_t2_2']['b'])
        al2 = jnp.array([p['a1'], p['a21'], p['a22'], 0, 0, 0, 0, 0], _f32)
        trp, a3, a4 = _S2(
            al2, tr, mask_e, p1, p2, cnt_pick, cnt_srce, g1, g2,
            wt, wp1, wp2, wm, bb,
            _pad2(p['l2_t1_1']['W'], F, 64), _padb(p['l2_t1_1']['b'], F),
            _pad2(p['l2_t2_1']['W'], F, 64), _padb(p['l2_t2_1']['b'], F),
            wh, bme)

        p3 = _PROP(a3, g1, idx_pick, am)
        p4 = _PROP(a4, g2, idx_srce, am)

        w21, w22 = p['l2_t1_2']['W'], p['l2_t2_2']['W']
        wt2 = jnp.zeros((F, 64), _f32)
        wt2 = wt2.at[0:15, 0:60].set(w21[:, 0:60]).at[15:30, 0:60].set(w22[:, 0:60])
        wp3 = jnp.zeros((F, F), _f32).at[0:15, 0:30].set(w21[:, 60:90])
        wp4 = jnp.zeros((F, F), _f32).at[15:30, 0:30].set(w22[:, 60:90])
        wm2 = jnp.zeros((F, FT), _f32)
        wm2 = wm2.at[0:15, 0:10].set(w21[:, 90:100]).at[15:30, 0:10].set(w22[:, 90:100])
        bb2 = jnp.zeros((1, F), _f32)
        bb2 = bb2.at[0, 0:15].set(p['l2_t1_2']['b']).at[0, 15:30].set(p['l2_t2_2']['b'])
        al3 = jnp.array([p['a2'], 0, 0, 0, 0, 0, 0, 0], _f32)
        (h,) = _S3(al3, trp, mask_e, p3, p4, cnt_pick, cnt_srce,
                   wt2, wp3, wp4, wm2, bb2)

    pbs, pbt = params['bip_src'], params['bip_sta']
    v_w, u_w = pbs['fc1_0']['W'], pbt['fc1_0']['W']
    c_src, c_sta = _SR1(
        h, mask_e, psta4, psrc4,
        _pad2(v_w[:, :30], F, F), _pad2(v_w[:, 30:40], F, FT),
        _pad2(v_w[:, 40:43], F, 4),
        _pad2(u_w[:, :30], F, F), _pad2(u_w[:, 30:40], F, FT),
        _pad2(u_w[:, 40:43], F, 4))

    srcsp = jnp.zeros((NSEG_SRC, 4), _f32).at[:srcs_cart.shape[0], 0:3].set(
        srcs_cart / 30000.0)
    locsp = jnp.zeros((NSEG_STA, 4), _f32).at[:locs_cart.shape[0], 0:3].set(
        locs_cart / 30000.0)
    d_src, d_sta = _SD(srcsp, locsp,
                       _pad2(v_w[:, 40:43], F, 4), _padb(pbs['fc1_0']['b'], F),
                       _pad2(u_w[:, 40:43], F, 4), _padb(pbt['fc1_0']['b'], F))

    m1_src = _EMAP_SRC(c_src, d_src, idx_rsrc, _avec(pbs['a_fc1']))
    m1_sta = _EMAP_STA(c_sta, d_sta, idx_rsta, _avec(pbt['a_fc1']))

    als = jnp.array([pbs['a1'], 0, 0, 0, 0, 0, 0, 0], _f32)
    alt = jnp.array([pbt['a1'], 0, 0, 0, 0, 0, 0, 0], _f32)
    (m2_src,) = _SR2(als, m1_src, _pad2(pbs['fc1_2']['W'], F, F),
                     _padb(pbs['fc1_2']['b'], F))
    (m2_sta,) = _SR2(alt, m1_sta, _pad2(pbt['fc1_2']['W'], F, F),
                     _padb(pbt['fc1_2']['b'], F))

    s_src = _RSCAT_SRC(m2_src, idx_rsrc)
    s_sta = _RSCAT_STA(m2_sta, idx_rsta)

    pm, md = params['proj_memory'], params['merge_data']
    pj, pt, pc = params['proj'], params['proj_t'], params['proj_c']
    memp = jnp.zeros((NSEG_SRC, 8), _f32).at[:memory.shape[0], 0:4].set(memory)
    alf = jnp.array([pbs['a2'], pbt['a2'], pm['a'], md['a'],
                     pj['a'], pt['a'], pc['a'], 0.0], _f32)
    md0 = md['layers'][0]['W']
    pred, pred_t, corr = _SF(
        alf, s_src, cnt_rsrc, s_sta, cnt_rsta, memp,
        _pad2(pbs['fc2']['W'], 16, F), _padb(pbs['fc2']['b'], 16),
        _pad2(pbt['fc2']['W'], 16, F), _padb(pbt['fc2']['b'], 16),
        _pad2(pm['layers'][0]['W'], F, 8), _padb(pm['layers'][0]['b'], F),
        _pad2(pm['layers'][1]['W'], 16, F), _padb(pm['layers'][1]['b'], 16),
        _pad2(md0[:, :15], F, 16), _pad2(md0[:, 15:30], F, 16),
        _padb(md['layers'][0]['b'], F),
        _pad2(md['layers'][1]['W'], F, F), _padb(md['layers'][1]['b'], F),
        _pad2(pj['layers'][0]['W'], F, F), _padb(pj['layers'][0]['b'], F),
        _pad2(pj['layers'][1]['W'], 8, F), _padb(pj['layers'][1]['b'], 8),
        _pad2(pt['layers'][0]['W'], 16, F), _padb(pt['layers'][0]['b'], 16),
        _pad2(pt['layers'][1]['W'], 8, 16), _padb(pt['layers'][1]['b'], 8),
        _pad2(pc['layers'][0]['W'], 16, 16), _padb(pc['layers'][0]['b'], 16),
        _pad2(pc['layers'][1]['W'], 8, 16), _padb(pc['layers'][1]['b'], 8))

    return (pred[:1000, :3], pred_t[:1000, :1], corr[:500, :2])


# speedup vs baseline: 11.1235x; 1.2653x over previous
import functools

import jax
import jax.numpy as jnp
from jax import lax
from jax.experimental import pallas as pl
from jax.experimental.pallas import tpu as pltpu
from jax.experimental.pallas import tpu_sc as plsc

NREAL = 50000
NPAD = 53248
F = 32
FT = 16
NE = 1600000
WSZ = 128
NWRK = 32
NC = 2
NSUB = 16
NW_E = 392
EPAD = NWRK * NW_E * WSZ
NW_R = 13
NSEG_SRC = 1024
NSEG_STA = 512
BM = 512
GRID = NPAD // BM

_f32 = jnp.float32


def _mesh():
    return plsc.VectorSubcoreMesh(core_axis_name="c", subcore_axis_name="s")


def _make_prop():
    @functools.partial(
        pl.kernel,
        compiler_params=pltpu.CompilerParams(use_tc_tiling_on_sc=False),
        out_type=jax.ShapeDtypeStruct((NC, NPAD, F), _f32),
        mesh=_mesh(),
        scratch_types=(
            [pltpu.VMEM((2 * WSZ,), jnp.int32) for _ in range(2)]
            + [pltpu.VMEM((WSZ,), jnp.int32) for _ in range(2)]
            + [pltpu.VMEM((WSZ, F // 2), jnp.int32) for _ in range(2)]
            + [pltpu.VMEM((WSZ, F // 2), jnp.int32) for _ in range(2)]
            + [pltpu.VMEM((WSZ, F), _f32) for _ in range(2)]
            + [pltpu.VMEM((16,), _f32),
               pltpu.VMEM((64, F), _f32),
               pltpu.VMEM_SHARED((NPAD, F), _f32)]
            + [pltpu.SemaphoreType.DMA for _ in range(6)]
        ),
    )
    def prop(a_hbm, g_hbm, idx_hbm, alpha_hbm, out_hbm,
             ib0, ib1, sb0, sb1, a0, a1, g0, g1, m0, m1,
             alphav, zbuf, acc, si0, si1, sg0, sg1, ss0, ss1):
        ibuf = (ib0, ib1)
        sibuf = (sb0, sb1)
        abuf = (a0, a1)
        gbuf = (g0, g1)
        mbuf = (m0, m1)
        si = (si0, si1)
        sg = (sg0, sg1)
        ss = (ss0, ss1)

        c = lax.axis_index("c")
        s = lax.axis_index("s")
        wid = s * NC + c
        rows_per_sub = NPAD // NSUB
        rows0 = s * rows_per_sub

        def zb(i, _):
            zbuf[i, pl.ds(0, 16)] = jnp.zeros((16,), _f32)
            zbuf[i, pl.ds(16, 16)] = jnp.zeros((16,), _f32)
            return 0
        lax.fori_loop(0, 64, zb, 0)

        def zc(k, _):
            pltpu.sync_copy(zbuf, acc.at[pl.ds(rows0 + k * 64, 64)])
            return 0
        lax.fori_loop(0, rows_per_sub // 64, zc, 0)

        pltpu.sync_copy(alpha_hbm, alphav)
        av = alphav[...]
        plsc.subcore_barrier()

        wbase = wid * NW_E

        def fire_idx(k, w):
            pltpu.async_copy(idx_hbm.at[wbase + w], ibuf[k], si[k])

        def wait_idx(k, w):
            pltpu.make_async_copy(idx_hbm.at[wbase + w], ibuf[k],
                                  si[k]).wait()

        def fire_gather(k):
            pltpu.async_copy(a_hbm.at[ibuf[k].at[pl.ds(0, WSZ)]],
                             abuf[k], sg[k])
            pltpu.async_copy(g_hbm.at[ibuf[k].at[pl.ds(WSZ, WSZ)]],
                             gbuf[k], sg[k])

        def wait_gather(k):
            pltpu.make_async_copy(a_hbm.at[ibuf[k].at[pl.ds(0, WSZ)]],
                                  abuf[k], sg[k]).wait()
            pltpu.make_async_copy(g_hbm.at[ibuf[k].at[pl.ds(WSZ, WSZ)]],
                                  gbuf[k], sg[k]).wait()

        def copy_scatter_idx(k):
            def cpb(i, _):
                sibuf[k][pl.ds(i * 16, 16)] = ibuf[k][pl.ds(WSZ + i * 16, 16)]
                return 0
            lax.fori_loop(0, WSZ // 16, cpb, 0, unroll=8)

        def fire_scat(k):
            pltpu.async_copy(mbuf[k], acc.at[sibuf[k]], ss[k], add=True)

        def wait_scat(k):
            pltpu.make_async_copy(mbuf[k], acc.at[sibuf[k]], ss[k]).wait()

        hi_mask = jnp.full((16,), -65536, jnp.int32)

        def compute(k):
            ab, gb, mf = abuf[k], gbuf[k], mbuf[k]

            def cb(i, _):
                wa = ab[i, :]
                wg = gb[i, :]
                a_e = lax.bitcast_convert_type(wa << 16, _f32)
                g_e = lax.bitcast_convert_type(wg << 16, _f32)
                a_o = lax.bitcast_convert_type(wa & hi_mask, _f32)
                g_o = lax.bitcast_convert_type(wg & hi_mask, _f32)
                d0 = a_e - g_e
                d1 = a_o - g_o
                mf[i, pl.ds(0, 16)] = (jnp.maximum(d0, 0.0)
                                       + av * jnp.minimum(d0, 0.0))
                mf[i, pl.ds(16, 16)] = (jnp.maximum(d1, 0.0)
                                        + av * jnp.minimum(d1, 0.0))
                return 0
            lax.fori_loop(0, WSZ, cb, 0, unroll=8)

        fire_idx(0, 0)
        fire_idx(1, 1)

        def body(j, _):
            w0 = 2 * j
            for k in range(2):
                wait_idx(k, w0 + k)

                @pl.when(j > 0)
                def _():
                    wait_scat(k)
                copy_scatter_idx(k)
                fire_gather(k)
            for k in range(2):
                wait_gather(k)

                @pl.when(w0 + k + 2 < NW_E)
                def _():
                    fire_idx(k, w0 + k + 2)
                compute(k)
                fire_scat(k)
            return 0
        lax.fori_loop(0, NW_E // 2, body, 0)
        wait_scat(0)
        wait_scat(1)

        plsc.subcore_barrier()
        pltpu.sync_copy(acc.at[pl.ds(rows0, rows_per_sub)],
                        out_hbm.at[c, pl.ds(rows0, rows_per_sub)])

    return prop


def _make_counts_big():
    rows_per_sub = NPAD // NSUB
    nb = 4

    @functools.partial(
        pl.kernel,
        compiler_params=pltpu.CompilerParams(use_tc_tiling_on_sc=False),
        out_type=jax.ShapeDtypeStruct((NC, NPAD), _f32),
        mesh=_mesh(),
        scratch_types=(
            [pltpu.VMEM((2 * WSZ,), jnp.int32) for _ in range(4)]
            + [pltpu.VMEM((WSZ,), jnp.int32) for _ in range(4)]
            + [pltpu.VMEM((WSZ,), _f32), pltpu.VMEM((WSZ,), _f32)]
            + [pltpu.SemaphoreType.DMA for _ in range(8)]
            + [pltpu.VMEM_SHARED((NPAD,), _f32)]
        ),
    )
    def counts(idx_hbm, out_hbm, *scr):
        ibuf = scr[0:4]
        sibuf = scr[4:8]
        ones, zv = scr[8:10]
        si = scr[10:14]
        ss = scr[14:18]
        acc = scr[18]

        c = lax.axis_index("c")
        s = lax.axis_index("s")
        wid = s * NC + c
        rows0 = s * rows_per_sub

        def ib(i, _):
            ones[pl.ds(i * 16, 16)] = jnp.full((16,), 1.0, _f32)
            zv[pl.ds(i * 16, 16)] = jnp.zeros((16,), _f32)
            return 0
        lax.fori_loop(0, WSZ // 16, ib, 0)

        def zc(k, _):
            pltpu.sync_copy(zv, acc.at[pl.ds(rows0 + k * WSZ, WSZ)])
            return 0
        lax.fori_loop(0, rows_per_sub // WSZ, zc, 0)
        plsc.subcore_barrier()

        wbase = wid * NW_E

        def fire_idx(k, w):
            pltpu.async_copy(idx_hbm.at[wbase + w], ibuf[k], si[k])

        def wait_idx(k, w):
            pltpu.make_async_copy(idx_hbm.at[wbase + w], ibuf[k],
                                  si[k]).wait()

        def copy_scatter_idx(k):
            def cpb(i, _):
                sibuf[k][pl.ds(i * 16, 16)] = ibuf[k][pl.ds(WSZ + i * 16, 16)]
                return 0
            lax.fori_loop(0, WSZ // 16, cpb, 0, unroll=8)

        def fire_scat(k):
            pltpu.async_copy(ones, acc.at[sibuf[k]], ss[k], add=True)

        def wait_scat(k):
            pltpu.make_async_copy(ones, acc.at[sibuf[k]], ss[k]).wait()

        for k in range(nb):
            fire_idx(k, k)

        def body(j, _):
            w0 = nb * j
            for k in range(nb):
                wait_idx(k, w0 + k)

                @pl.when(j > 0)
                def _():
                    wait_scat(k)
                copy_scatter_idx(k)

                @pl.when(w0 + k + nb < NW_E)
                def _():
                    fire_idx(k, w0 + k + nb)
                fire_scat(k)
            return 0
        lax.fori_loop(0, NW_E // nb, body, 0)
        for k in range(nb):
            wait_scat(k)

        plsc.subcore_barrier()
        pltpu.sync_copy(acc.at[pl.ds(rows0, rows_per_sub)],
                        out_hbm.at[c, pl.ds(rows0, rows_per_sub)])

    return counts


def _make_counts(nseg, nwin):
    rows_per_sub = nseg // NSUB

    @functools.partial(
        pl.kernel,
        compiler_params=pltpu.CompilerParams(use_tc_tiling_on_sc=False),
        out_type=jax.ShapeDtypeStruct((NC, nseg), _f32),
        mesh=_mesh(),
        scratch_types=[
            pltpu.VMEM((2, WSZ), jnp.int32),
            pltpu.VMEM((WSZ,), _f32),
            pltpu.VMEM_SHARED((nseg,), _f32),
        ],
    )
    def counts(idx_hbm, out_hbm, idxb, ones, acc):
        c = lax.axis_index("c")
        s = lax.axis_index("s")
        wid = s * NC + c
        rows0 = s * rows_per_sub

        def zb(i, _):
            ones[pl.ds(i * 16, 16)] = jnp.zeros((16,), _f32)
            return 0
        lax.fori_loop(0, WSZ // 16, zb, 0)
        if rows_per_sub <= WSZ:
            pltpu.sync_copy(ones.at[pl.ds(0, rows_per_sub)],
                            acc.at[pl.ds(rows0, rows_per_sub)])
        else:
            def zc(k, _):
                pltpu.sync_copy(ones, acc.at[pl.ds(rows0 + k * WSZ, WSZ)])
                return 0
            lax.fori_loop(0, rows_per_sub // WSZ, zc, 0)

        def ob(i, _):
            ones[pl.ds(i * 16, 16)] = jnp.full((16,), 1.0, _f32)
            return 0
        lax.fori_loop(0, WSZ // 16, ob, 0)
        plsc.subcore_barrier()

        wbase = wid * nwin

        def body(w, _):
            pltpu.sync_copy(idx_hbm.at[wbase + w], idxb)
            pltpu.sync_copy(ones, acc.at[idxb.at[1]], add=True)
            return 0
        lax.fori_loop(0, nwin, body, 0)

        plsc.subcore_barrier()
        pltpu.sync_copy(acc.at[pl.ds(rows0, rows_per_sub)],
                        out_hbm.at[c, pl.ds(rows0, rows_per_sub)])

    return counts


def _make_tab(ntab):
    @functools.partial(
        pl.kernel,
        compiler_params=pltpu.CompilerParams(use_tc_tiling_on_sc=False),
        out_type=jax.ShapeDtypeStruct((NPAD, FT), _f32),
        mesh=_mesh(),
        scratch_types=[
            pltpu.VMEM((WSZ,), jnp.int32),
            pltpu.VMEM((WSZ, FT), _f32),
            pltpu.SemaphoreType.DMA,
        ],
    )
    def tab(t_hbm, idx_hbm, out_hbm, idxb, tbuf, sem):
        c = lax.axis_index("c")
        s = lax.axis_index("s")
        wid = s * NC + c
        ebase = wid * NW_R * WSZ

        def body(w, _):
            pltpu.sync_copy(idx_hbm.at[wid * NW_R + w], idxb)
            pltpu.async_copy(t_hbm.at[idxb], tbuf, sem).wait()
            pltpu.sync_copy(tbuf, out_hbm.at[pl.ds(ebase + w * WSZ, WSZ)])
            return 0
        lax.fori_loop(0, NW_R, body, 0)

    return tab


def _make_emap(nseg):
    @functools.partial(
        pl.kernel,
        compiler_params=pltpu.CompilerParams(use_tc_tiling_on_sc=False),
        out_type=jax.ShapeDtypeStruct((NPAD, F), _f32),
        mesh=_mesh(),
        scratch_types=[
            pltpu.VMEM((2, WSZ), jnp.int32),
            pltpu.VMEM((WSZ, F), _f32),
            pltpu.VMEM((WSZ, F), _f32),
            pltpu.VMEM((16,), _f32),
            pltpu.SemaphoreType.DMA,
            pltpu.SemaphoreType.DMA,
        ],
    )
    def emap(c_hbm, d_hbm, idx_hbm, alpha_hbm, out_hbm,
             idxb, cbuf, dbuf, alphav, sc_, sd_):
        c = lax.axis_index("c")
        s = lax.axis_index("s")
        wid = s * NC + c
        ebase = wid * NW_R * WSZ
        pltpu.sync_copy(alpha_hbm, alphav)
        av = alphav[...]

        def body(w, _):
            pltpu.sync_copy(idx_hbm.at[wid * NW_R + w], idxb)
            pltpu.async_copy(c_hbm.at[idxb.at[0]], cbuf, sc_)
            pltpu.async_copy(d_hbm.at[idxb.at[1]], dbuf, sd_)
            pltpu.make_async_copy(c_hbm.at[idxb.at[0]], cbuf, sc_).wait()
            pltpu.make_async_copy(d_hbm.at[idxb.at[1]], dbuf, sd_).wait()

            def cb(i, _):
                for hh in (0, 16):
                    d = cbuf[i, pl.ds(hh, 16)] + dbuf[i, pl.ds(hh, 16)]
                    cbuf[i, pl.ds(hh, 16)] = (jnp.maximum(d, 0.0)
                                              + av * jnp.minimum(d, 0.0))
                return 0
            lax.fori_loop(0, WSZ, cb, 0, unroll=4)
            pltpu.sync_copy(cbuf, out_hbm.at[pl.ds(ebase + w * WSZ, WSZ)])
            return 0
        lax.fori_loop(0, NW_R, body, 0)

    return emap


def _make_rscatter(nseg):
    rows_per_sub = nseg // NSUB

    @functools.partial(
        pl.kernel,
        compiler_params=pltpu.CompilerParams(use_tc_tiling_on_sc=False),
        out_type=jax.ShapeDtypeStruct((NC, nseg, F), _f32),
        mesh=_mesh(),
        scratch_types=[
            pltpu.VMEM((2, WSZ), jnp.int32),
            pltpu.VMEM((WSZ, F), _f32),
            pltpu.VMEM((WSZ, F), _f32),
            pltpu.VMEM_SHARED((nseg, F), _f32),
            pltpu.SemaphoreType.DMA,
        ],
    )
    def rscatter(m_hbm, idx_hbm, out_hbm, idxb, mbuf, zbuf, acc, sem):
        c = lax.axis_index("c")
        s = lax.axis_index("s")
        wid = s * NC + c
        rows0 = s * rows_per_sub
        ebase = wid * NW_R * WSZ

        def zb(i, _):
            zbuf[i, pl.ds(0, 16)] = jnp.zeros((16,), _f32)
            zbuf[i, pl.ds(16, 16)] = jnp.zeros((16,), _f32)
            return 0
        lax.fori_loop(0, WSZ, zb, 0)
        pltpu.sync_copy(zbuf.at[pl.ds(0, rows_per_sub)],
                        acc.at[pl.ds(rows0, rows_per_sub)])
        plsc.subcore_barrier()

        def body(w, _):
            pltpu.sync_copy(idx_hbm.at[wid * NW_R + w], idxb)
            pltpu.async_copy(m_hbm.at[pl.ds(ebase + w * WSZ, WSZ)],
                             mbuf, sem).wait()
            pltpu.sync_copy(mbuf, acc.at[idxb.at[1]], add=True)
            return 0
        lax.fori_loop(0, NW_R, body, 0)

        plsc.subcore_barrier()
        pltpu.sync_copy(acc.at[pl.ds(rows0, rows_per_sub)],
                        out_hbm.at[c, pl.ds(rows0, rows_per_sub)])

    return rscatter


_PROP = _make_prop()
_COUNTS_BIG = _make_counts_big()
_COUNTS_SRC = _make_counts(NSEG_SRC, NW_R)
_COUNTS_STA = _make_counts(NSEG_STA, NW_R)
_TAB_SRC = _make_tab(NSEG_SRC)
_TAB_STA = _make_tab(NSEG_STA)
_EMAP_SRC = _make_emap(NSEG_SRC)
_EMAP_STA = _make_emap(NSEG_STA)
_RSCAT_SRC = _make_rscatter(NSEG_SRC)
_RSCAT_STA = _make_rscatter(NSEG_STA)


def _prelu(x, a):
    return jnp.where(x >= 0, x, a * x)


def _row_spec(f):
    return pl.BlockSpec((BM, f), lambda i: (i, 0))


def _stack_spec(f):
    return pl.BlockSpec((2, BM, f), lambda i: (0, i, 0))


def _full_spec(shape):
    nd = len(shape)
    return pl.BlockSpec(shape, lambda i: (0,) * nd)


def _smem_spec():
    return pl.BlockSpec(memory_space=pltpu.SMEM)


def _s0_body(al, xp, mk, tg, e0m, e0g, b0, e1, b1, sx, sg, me_o, h0_o):
    a_e = al[0]
    t8 = tg[...][:, 0:8]
    h = _prelu(mk[...] @ e0m[...].T + t8 @ e0g[...].T + b0[...], a_e)
    me_o[...] = h @ e1[...].T + b1[...]
    h0_o[...] = xp[...] @ sx[...].T + t8 @ sg[...].T


def _s1_body(al, h, me, psta, psrc, wih, wim, bi, wh, bme, we3,
             tr_o, g1_o, g2_o, a1_o, a2_o, g1b_o, g2b_o):
    a_init, a11, a12 = al[0], al[1], al[2]
    tr = _prelu(h[...] @ wih[...].T + me[...] @ wim[...].T + bi[...], a_init)
    g1 = psta[...] @ we3[...].T
    g2 = psrc[...] @ we3[...].T
    tr_o[...] = tr
    a1_o[...] = (_prelu(tr, a11) @ wh[...].T + bme[...] + g1).astype(jnp.bfloat16)
    a2_o[...] = (_prelu(tr, a12) @ wh[...].T + bme[...] + g2).astype(jnp.bfloat16)
    g1_o[...] = g1
    g2_o[...] = g2
    g1b_o[...] = g1.astype(jnp.bfloat16)
    g2b_o[...] = g2.astype(jnp.bfloat16)


def _s2_body(al, tr, me, p1, p2, c1, c2, g1, g2,
             wt, wp1, wp2, wm, bb, w21, b21, w22, b22, wh, bme,
             trp_o, a3_o, a4_o):
    a1s, a21, a22 = al[0], al[1], al[2]
    ic1 = 1.0 / jnp.maximum(c1[...][0] + c1[...][1], 1.0)
    ic2 = 1.0 / jnp.maximum(c2[...][0] + c2[...][1], 1.0)
    prop1 = (p1[...][0] + p1[...][1]) * ic1
    prop2 = (p2[...][0] + p2[...][1]) * ic2
    trp = _prelu(tr[...] @ wt[...].T + prop1 @ wp1[...].T
                 + prop2 @ wp2[...].T + me[...] @ wm[...].T + bb[...], a1s)
    u1 = _prelu(trp @ w21[...].T + b21[...], a21)
    u2 = _prelu(trp @ w22[...].T + b22[...], a22)
    trp_o[...] = trp
    a3_o[...] = (u1 @ wh[...].T + bme[...] + g1[...]).astype(jnp.bfloat16)
    a4_o[...] = (u2 @ wh[...].T + bme[...] + g2[...]).astype(jnp.bfloat16)


def _s3_body(al, trp, me, p3, p4, c1, c2, wt, wp1, wp2, wm, bb, h_o):
    a2s = al[0]
    ic1 = 1.0 / jnp.maximum(c1[...][0] + c1[...][1], 1.0)
    ic2 = 1.0 / jnp.maximum(c2[...][0] + c2[...][1], 1.0)
    prop3 = (p3[...][0] + p3[...][1]) * ic1
    prop4 = (p4[...][0] + p4[...][1]) * ic2
    h_o[...] = _prelu(trp[...] @ wt[...].T + prop3 @ wp1[...].T
                      + prop4 @ wp2[...].T + me[...] @ wm[...].T + bb[...], a2s)


def _sr1_body(h, me, psta, psrc, v1, v2, v3, u1, u2, u3, cs_o, ct_o):
    cs_o[...] = h[...] @ v1[...].T + me[...] @ v2[...].T - psta[...] @ v3[...].T
    ct_o[...] = h[...] @ u1[...].T + me[...] @ u2[...].T - psrc[...] @ u3[...].T


def _sr2_body(al, m1, w, b, o):
    o[...] = _prelu(m1[...] @ w[...].T + b[...], al[0])


def _sd_body(srcsp, locsp, v3, b1s, u3, b1t, ds_o, dt_o):
    ds_o[...] = srcsp[...] @ v3[...].T + b1s[...]
    dt_o[...] = locsp[...] @ u3[...].T + b1t[...]


def _sf_body(al, ssrc, cs, ssta, ct, memp,
             fw2s, fb2s, fw2t, fb2t, pmw0, pmb0, pmw1, pmb1,
             mdw0a, mdw0b, mdb0, mdw1, mdb1,
             pw0, pb0, pw1, pb1, tw0, tb0, tw1, tb1, cw0, cb0, cw1, cb1,
             pred_o, predt_o, corr_o):
    a2s, a2t, apm, amd, apj, apt, apc = (al[0], al[1], al[2], al[3],
                                         al[4], al[5], al[6])
    invs = 1.0 / jnp.maximum(cs[...][0] + cs[...][1], 1.0)
    aggs = (ssrc[...][0] + ssrc[...][1]) * invs
    semb = _prelu(aggs @ fw2s[...].T + fb2s[...], a2s)
    mp = _prelu(memp[...] @ pmw0[...].T + pmb0[...], apm) @ pmw1[...].T + pmb1[...]
    mer = (_prelu(semb @ mdw0a[...].T + mp @ mdw0b[...].T + mdb0[...], amd)
           @ mdw1[...].T + mdb1[...])
    pred_o[...] = (_prelu(mer @ pw0[...].T + pb0[...], apj)
                   @ pw1[...].T + pb1[...]) * 5000.0
    predt_o[...] = (_prelu(mer @ tw0[...].T + tb0[...], apt)
                    @ tw1[...].T + tb1[...])
    invt = 1.0 / jnp.maximum(ct[...][0] + ct[...][1], 1.0)
    aggt = (ssta[...][0] + ssta[...][1]) * invt
    temb = _prelu(aggt @ fw2t[...].T + fb2t[...], a2t)
    corr_o[...] = (_prelu(temb @ cw0[...].T + cb0[...], apc)
                   @ cw1[...].T + cb1[...])


def _shape(n, f):
    return jax.ShapeDtypeStruct((n, f), _f32)


_S0 = pl.pallas_call(
    _s0_body,
    grid=(GRID,),
    in_specs=[_smem_spec(), _row_spec(24), _row_spec(24), _row_spec(FT),
              _full_spec((32, 24)), _full_spec((32, 8)), _full_spec((1, 32)),
              _full_spec((16, 32)), _full_spec((1, 16)),
              _full_spec((32, 24)), _full_spec((32, 8))],
    out_specs=[_row_spec(FT), _row_spec(F)],
    out_shape=[_shape(NPAD, FT), _shape(NPAD, F)],
)

_S1 = pl.pallas_call(
    _s1_body,
    grid=(GRID,),
    in_specs=[_smem_spec(), _row_spec(F), _row_spec(FT), _row_spec(4),
              _row_spec(4),
              _full_spec((F, F)), _full_spec((F, FT)), _full_spec((1, F)),
              _full_spec((F, F)), _full_spec((1, F)), _full_spec((F, 4))],
    out_specs=[_row_spec(F)] * 7,
    out_shape=[_shape(NPAD, F)] * 3
    + [jax.ShapeDtypeStruct((NPAD, F), jnp.bfloat16)] * 4,
)

_S2 = pl.pallas_call(
    _s2_body,
    grid=(GRID,),
    in_specs=[_smem_spec(), _row_spec(F), _row_spec(FT),
              _stack_spec(F), _stack_spec(F),
              _stack_spec(1), _stack_spec(1),
              _row_spec(F), _row_spec(F),
              _full_spec((64, F)), _full_spec((64, F)), _full_spec((64, F)),
              _full_spec((64, FT)), _full_spec((1, 64)),
              _full_spec((F, 64)), _full_spec((1, F)),
              _full_spec((F, 64)), _full_spec((1, F)),
              _full_spec((F, F)), _full_spec((1, F))],
    out_specs=[_row_spec(64), _row_spec(F), _row_spec(F)],
    out_shape=[_shape(NPAD, 64),
               jax.ShapeDtypeStruct((NPAD, F), jnp.bfloat16),
               jax.ShapeDtypeStruct((NPAD, F), jnp.bfloat16)],
)

_S3 = pl.pallas_call(
    _s3_body,
    grid=(GRID,),
    in_specs=[_smem_spec(), _row_spec(64), _row_spec(FT),
              _stack_spec(F), _stack_spec(F),
              _stack_spec(1), _stack_spec(1),
              _full_spec((F, 64)), _full_spec((F, F)), _full_spec((F, F)),
              _full_spec((F, FT)), _full_spec((1, F))],
    out_specs=[_row_spec(F)],
    out_shape=[_shape(NPAD, F)],
)

_SR1 = pl.pallas_call(
    _sr1_body,
    grid=(GRID,),
    in_specs=[_row_spec(F), _row_spec(FT), _row_spec(4), _row_spec(4),
              _full_spec((F, F)), _full_spec((F, FT)), _full_spec((F, 4)),
              _full_spec((F, F)), _full_spec((F, FT)), _full_spec((F, 4))],
    out_specs=[_row_spec(F), _row_spec(F)],
    out_shape=[_shape(NPAD, F), _shape(NPAD, F)],
)

_SR2 = pl.pallas_call(
    _sr2_body,
    grid=(GRID,),
    in_specs=[_smem_spec(), _row_spec(F),
              _full_spec((F, F)), _full_spec((1, F))],
    out_specs=[_row_spec(F)],
    out_shape=[_shape(NPAD, F)],
)

_SD = pl.pallas_call(
    _sd_body,
    grid=(1,),
    in_specs=[_full_spec((NSEG_SRC, 4)), _full_spec((NSEG_STA, 4)),
              _full_spec((F, 4)), _full_spec((1, F)),
              _full_spec((F, 4)), _full_spec((1, F))],
    out_specs=[_full_spec((NSEG_SRC, F)), _full_spec((NSEG_STA, F))],
    out_shape=[_shape(NSEG_SRC, F), _shape(NSEG_STA, F)],
)

_SF = pl.pallas_call(
    _sf_body,
    grid=(1,),
    in_specs=[_smem_spec(),
              _full_spec((NC, NSEG_SRC, F)), _full_spec((NC, NSEG_SRC, 1)),
              _full_spec((NC, NSEG_STA, F)), _full_spec((NC, NSEG_STA, 1)),
              _full_spec((NSEG_SRC, 8)),
              _full_spec((16, F)), _full_spec((1, 16)),
              _full_spec((16, F)), _full_spec((1, 16)),
              _full_spec((F, 8)), _full_spec((1, F)),
              _full_spec((16, F)), _full_spec((1, 16)),
              _full_spec((F, 16)), _full_spec((F, 16)), _full_spec((1, F)),
              _full_spec((F, F)), _full_spec((1, F)),
              _full_spec((F, F)), _full_spec((1, F)),
              _full_spec((8, F)), _full_spec((1, 8)),
              _full_spec((16, F)), _full_spec((1, 16)),
              _full_spec((8, 16)), _full_spec((1, 8)),
              _full_spec((16, 16)), _full_spec((1, 16)),
              _full_spec((8, 16)), _full_spec((1, 8))],
    out_specs=[_full_spec((NSEG_SRC, 8)), _full_spec((NSEG_SRC, 8)),
               _full_spec((NSEG_STA, 8))],
    out_shape=[_shape(NSEG_SRC, 8), _shape(NSEG_SRC, 8), _shape(NSEG_STA, 8)],
)


_PERM = tuple(list(range(0, F, 2)) + list(range(1, F, 2)))


def _i32rows(x):
    return jax.lax.bitcast_convert_type(
        x.reshape(NPAD, F // 2, 2), jnp.int32)


def _pad2(w, r, c):
    return jnp.zeros((r, c), _f32).at[:w.shape[0], :w.shape[1]].set(w)


def _padb(b, c):
    return jnp.zeros((1, c), _f32).at[0, :b.shape[0]].set(b)


def _pad_nodes(x, cols):
    return jnp.zeros((NPAD, cols), _f32).at[:x.shape[0], :x.shape[1]].set(x)


def _prep_edges(src, dst, nwin, pad_base, pad_mod):
    ne = src.shape[0]
    npad = NWRK * nwin * WSZ - ne
    srcp = jnp.concatenate([src, jnp.zeros((npad,), jnp.int32)])
    dstp = jnp.concatenate(
        [dst, pad_base + (jnp.arange(npad, dtype=jnp.int32) % pad_mod)])
    idx = jnp.stack([srcp.reshape(NWRK, nwin, WSZ),
                     dstp.reshape(NWRK, nwin, WSZ)], axis=2)
    return idx.reshape(NWRK * nwin, 2, WSZ)


def _prep_edges_big(src, dst, pad_base, pad_mod):
    ne = src.shape[0]
    npad = EPAD - ne
    srcp = jnp.concatenate([src, jnp.zeros((npad,), jnp.int32)])
    dstp = jnp.concatenate(
        [dst, pad_base + (jnp.arange(npad, dtype=jnp.int32) % pad_mod)])
    idx = jnp.stack([srcp.reshape(NWRK, NW_E, WSZ),
                     dstp.reshape(NWRK, NW_E, WSZ)], axis=2)
    return idx.reshape(NWRK * NW_E, 2 * WSZ)


def _prep_node_idx(idx):
    p = jnp.zeros((NPAD,), jnp.int32).at[:idx.shape[0]].set(idx)
    return p.reshape(NWRK * NW_R, WSZ)


def _avec(a):
    return jnp.full((16,), a, _f32)


def kernel(x, mask, A_in_pick, A_in_src, A_src_in_product, A_sta_in_product,
           A_src_in_sta, locs_cart, srcs_cart, memory, params):
    sta_id = A_src_in_sta[0]
    src_id = A_src_in_sta[1]

    t_src = jnp.zeros((NSEG_SRC, FT), _f32)
    t_src = t_src.at[:memory.shape[0], 0:4].set(memory)
    t_src = t_src.at[:srcs_cart.shape[0], 4:7].set(srcs_cart / 30000.0)
    t_sta = jnp.zeros((NSEG_STA, FT), _f32)
    t_sta = t_sta.at[:locs_cart.shape[0], 0:3].set(locs_cart / 30000.0)

    idx_src_nodes = _prep_node_idx(src_id)
    idx_sta_nodes = _prep_node_idx(sta_id)
    idx_pick = _prep_edges_big(A_in_pick[0], A_in_pick[1], NREAL, 1024)
    idx_srce = _prep_edges_big(A_in_src[0], A_in_src[1], NREAL, 1024)
    idx_rsrc = _prep_edges(A_src_in_product[0], A_src_in_product[1], NW_R,
                           1000, 24)
    idx_rsta = _prep_edges(A_sta_in_product[0], A_sta_in_product[1], NW_R,
                           500, 12)

    xp = _pad_nodes(x, 24)
    maskp = _pad_nodes(mask, 24)

    tg_src = _TAB_SRC(t_src, idx_src_nodes)
    tg_sta = _TAB_STA(t_sta, idx_sta_nodes)
    cnt_pick = _COUNTS_BIG(idx_pick).reshape(NC, NPAD, 1)
    cnt_srce = _COUNTS_BIG(idx_srce).reshape(NC, NPAD, 1)
    cnt_rsrc = _COUNTS_SRC(idx_rsrc).reshape(NC, NSEG_SRC, 1)
    cnt_rsta = _COUNTS_STA(idx_rsta).reshape(NC, NSEG_STA, 1)

    psta4 = tg_sta[:, 0:4]
    psrc4 = tg_src[:, 4:8]

    pe = params['embed_inpt']
    e0 = pe['layers'][0]['W']
    e1 = pe['layers'][1]['W']
    sx = jnp.zeros((F, 24), _f32).at[:18, :18].set(jnp.eye(18))
    sg = jnp.zeros((F, 8), _f32).at[18:22, 0:4].set(jnp.eye(4))
    al0 = jnp.array([pe['a']] + [0.0] * 7, _f32)
    mask_e, h = _S0(al0, xp, maskp, tg_src,
                    _pad2(e0[:, :18], 32, 24), _pad2(e0[:, 18:22], 32, 8),
                    _padb(pe['layers'][0]['b'], 32),
                    _pad2(e1, 16, 32), _padb(pe['layers'][1]['b'], 16),
                    sx, sg)

    for name in ['da1', 'da2', 'da3', 'da4', 'da5']:
        p = params[name]
        We = p['merge_edges']['W']
        wh = _pad2(We[:, :30], F, F)
        we3 = _pad2(We[:, 30:33], F, 4)
        bme = _padb(p['merge_edges']['b'], F)
        nin = 22 if name == 'da1' else 30
        wi = p['init_trns']['W']
        al1 = jnp.array([p['a_init'], p['a11'], p['a12'], 0, 0, 0, 0, 0], _f32)
        tr, g1, g2, a1, a2, g1b, g2b = _S1(
            al1, h, mask_e, psta4, psrc4,
            _pad2(wi[:, :nin], F, F), _pad2(wi[:, nin:nin + 10], F, FT),
            _padb(p['init_trns']['b'], F), wh, bme, we3)

        am = _avec(p['a_merge'])
        p1 = _PROP(_i32rows(a1), _i32rows(g1b), idx_pick, am)
        p2 = _PROP(_i32rows(a2), _i32rows(g2b), idx_srce, am)

        w11, w12 = p['l1_t1_2']['W'], p['l1_t2_2']['W']
        wt = jnp.zeros((64, F), _f32)
        wt = wt.at[0:30, 0:30].set(w11[:, 0:30]).at[30:60, 0:30].set(w12[:, 0:30])
        wp1 = jnp.zeros((64, F), _f32).at[0:30, 0:30].set(w11[:, 30:60])[:, _PERM]
        wp2 = jnp.zeros((64, F), _f32).at[30:60, 0:30].set(w12[:, 30:60])[:, _PERM]
        wm = jnp.zeros((64, FT), _f32)
        wm = wm.at[0:30, 0:10].set(w11[:, 60:70]).at[30:60, 0:10].set(w12[:, 60:70])
        bb = jnp.zeros((1, 64), _f32)
        bb = bb.at[0, 0:30].set(p['l1_t1_2']['b']).at[0, 30:60].set(p['l1_t2_2']['b'])
        al2 = jnp.array([p['a1'], p['a21'], p['a22'], 0, 0, 0, 0, 0], _f32)
        trp, a3, a4 = _S2(
            al2, tr, mask_e, p1, p2, cnt_pick, cnt_srce, g1, g2,
            wt, wp1, wp2, wm, bb,
            _pad2(p['l2_t1_1']['W'], F, 64), _padb(p['l2_t1_1']['b'], F),
            _pad2(p['l2_t2_1']['W'], F, 64), _padb(p['l2_t2_1']['b'], F),
            wh, bme)

        p3 = _PROP(_i32rows(a3), _i32rows(g1b), idx_pick, am)
        p4 = _PROP(_i32rows(a4), _i32rows(g2b), idx_srce, am)

        w21, w22 = p['l2_t1_2']['W'], p['l2_t2_2']['W']
        wt2 = jnp.zeros((F, 64), _f32)
        wt2 = wt2.at[0:15, 0:60].set(w21[:, 0:60]).at[15:30, 0:60].set(w22[:, 0:60])
        wp3 = jnp.zeros((F, F), _f32).at[0:15, 0:30].set(w21[:, 60:90])[:, _PERM]
        wp4 = jnp.zeros((F, F), _f32).at[15:30, 0:30].set(w22[:, 60:90])[:, _PERM]
        wm2 = jnp.zeros((F, FT), _f32)
        wm2 = wm2.at[0:15, 0:10].set(w21[:, 90:100]).at[15:30, 0:10].set(w22[:, 90:100])
        bb2 = jnp.zeros((1, F), _f32)
        bb2 = bb2.at[0, 0:15].set(p['l2_t1_2']['b']).at[0, 15:30].set(p['l2_t2_2']['b'])
        al3 = jnp.array([p['a2'], 0, 0, 0, 0, 0, 0, 0], _f32)
        (h,) = _S3(al3, trp, mask_e, p3, p4, cnt_pick, cnt_srce,
                   wt2, wp3, wp4, wm2, bb2)

    pbs, pbt = params['bip_src'], params['bip_sta']
    v_w, u_w = pbs['fc1_0']['W'], pbt['fc1_0']['W']
    c_src, c_sta = _SR1(
        h, mask_e, psta4, psrc4,
        _pad2(v_w[:, :30], F, F), _pad2(v_w[:, 30:40], F, FT),
        _pad2(v_w[:, 40:43], F, 4),
        _pad2(u_w[:, :30], F, F), _pad2(u_w[:, 30:40], F, FT),
        _pad2(u_w[:, 40:43], F, 4))

    srcsp = jnp.zeros((NSEG_SRC, 4), _f32).at[:srcs_cart.shape[0], 0:3].set(
        srcs_cart / 30000.0)
    locsp = jnp.zeros((NSEG_STA, 4), _f32).at[:locs_cart.shape[0], 0:3].set(
        locs_cart / 30000.0)
    d_src, d_sta = _SD(srcsp, locsp,
                       _pad2(v_w[:, 40:43], F, 4), _padb(pbs['fc1_0']['b'], F),
                       _pad2(u_w[:, 40:43], F, 4), _padb(pbt['fc1_0']['b'], F))

    m1_src = _EMAP_SRC(c_src, d_src, idx_rsrc, _avec(pbs['a_fc1']))
    m1_sta = _EMAP_STA(c_sta, d_sta, idx_rsta, _avec(pbt['a_fc1']))

    als = jnp.array([pbs['a1'], 0, 0, 0, 0, 0, 0, 0], _f32)
    alt = jnp.array([pbt['a1'], 0, 0, 0, 0, 0, 0, 0], _f32)
    (m2_src,) = _SR2(als, m1_src, _pad2(pbs['fc1_2']['W'], F, F),
                     _padb(pbs['fc1_2']['b'], F))
    (m2_sta,) = _SR2(alt, m1_sta, _pad2(pbt['fc1_2']['W'], F, F),
                     _padb(pbt['fc1_2']['b'], F))

    s_src = _RSCAT_SRC(m2_src, idx_rsrc)
    s_sta = _RSCAT_STA(m2_sta, idx_rsta)

    pm, md = params['proj_memory'], params['merge_data']
    pj, pt, pc = params['proj'], params['proj_t'], params['proj_c']
    memp = jnp.zeros((NSEG_SRC, 8), _f32).at[:memory.shape[0], 0:4].set(memory)
    alf = jnp.array([pbs['a2'], pbt['a2'], pm['a'], md['a'],
                     pj['a'], pt['a'], pc['a'], 0.0], _f32)
    md0 = md['layers'][0]['W']
    pred, pred_t, corr = _SF(
        alf, s_src, cnt_rsrc, s_sta, cnt_rsta, memp,
        _pad2(pbs['fc2']['W'], 16, F), _padb(pbs['fc2']['b'], 16),
        _pad2(pbt['fc2']['W'], 16, F), _padb(pbt['fc2']['b'], 16),
        _pad2(pm['layers'][0]['W'], F, 8), _padb(pm['layers'][0]['b'], F),
        _pad2(pm['layers'][1]['W'], 16, F), _padb(pm['layers'][1]['b'], 16),
        _pad2(md0[:, :15], F, 16), _pad2(md0[:, 15:30], F, 16),
        _padb(md['layers'][0]['b'], F),
        _pad2(md['layers'][1]['W'], F, F), _padb(md['layers'][1]['b'], F),
        _pad2(pj['layers'][0]['W'], F, F), _padb(pj['layers'][0]['b'], F),
        _pad2(pj['layers'][1]['W'], 8, F), _padb(pj['layers'][1]['b'], 8),
        _pad2(pt['layers'][0]['W'], 16, F), _padb(pt['layers'][0]['b'], 16),
        _pad2(pt['layers'][1]['W'], 8, 16), _padb(pt['layers'][1]['b'], 8),
        _pad2(pc['layers'][0]['W'], 16, 16), _padb(pc['layers'][0]['b'], 16),
        _pad2(pc['layers'][1]['W'], 8, 16), _padb(pc['layers'][1]['b'], 8))

    return (pred[:1000, :3], pred_t[:1000, :1], corr[:500, :2])


# speedup vs baseline: 11.5996x; 1.0428x over previous
import functools

import jax
import jax.numpy as jnp
from jax import lax
from jax.experimental import pallas as pl
from jax.experimental.pallas import tpu as pltpu
from jax.experimental.pallas import tpu_sc as plsc

NREAL = 50000
NPAD = 53248
F = 32
FT = 16
NE = 1600000
WSZ = 128
NWRK = 32
NC = 2
NSUB = 16
NW_E = 392
EPAD = NWRK * NW_E * WSZ
NW_R = 13
NSEG_SRC = 1024
NSEG_STA = 512
BM = 2048
GRID = NPAD // BM

_f32 = jnp.float32


def _mesh():
    return plsc.VectorSubcoreMesh(core_axis_name="c", subcore_axis_name="s")


def _make_prop():
    @functools.partial(
        pl.kernel,
        compiler_params=pltpu.CompilerParams(use_tc_tiling_on_sc=False),
        out_type=jax.ShapeDtypeStruct((NC, NPAD, F), _f32),
        mesh=_mesh(),
        scratch_types=(
            [pltpu.VMEM((2 * WSZ,), jnp.int32) for _ in range(2)]
            + [pltpu.VMEM((WSZ,), jnp.int32) for _ in range(2)]
            + [pltpu.VMEM((WSZ, F // 2), jnp.int32) for _ in range(2)]
            + [pltpu.VMEM((WSZ, F // 2), jnp.int32) for _ in range(2)]
            + [pltpu.VMEM((WSZ, F), _f32) for _ in range(2)]
            + [pltpu.VMEM((16,), _f32),
               pltpu.VMEM((64, F), _f32),
               pltpu.VMEM_SHARED((NPAD, F), _f32)]
            + [pltpu.SemaphoreType.DMA for _ in range(6)]
        ),
    )
    def prop(a_hbm, g_hbm, idx_hbm, alpha_hbm, out_hbm,
             ib0, ib1, sb0, sb1, a0, a1, g0, g1, m0, m1,
             alphav, zbuf, acc, si0, si1, sg0, sg1, ss0, ss1):
        ibuf = (ib0, ib1)
        sibuf = (sb0, sb1)
        abuf = (a0, a1)
        gbuf = (g0, g1)
        mbuf = (m0, m1)
        si = (si0, si1)
        sg = (sg0, sg1)
        ss = (ss0, ss1)

        c = lax.axis_index("c")
        s = lax.axis_index("s")
        wid = s * NC + c
        rows_per_sub = NPAD // NSUB
        rows0 = s * rows_per_sub

        def zb(i, _):
            zbuf[i, pl.ds(0, 16)] = jnp.zeros((16,), _f32)
            zbuf[i, pl.ds(16, 16)] = jnp.zeros((16,), _f32)
            return 0
        lax.fori_loop(0, 64, zb, 0)

        def zc(k, _):
            pltpu.sync_copy(zbuf, acc.at[pl.ds(rows0 + k * 64, 64)])
            return 0
        lax.fori_loop(0, rows_per_sub // 64, zc, 0)

        pltpu.sync_copy(alpha_hbm, alphav)
        av = alphav[...]
        plsc.subcore_barrier()

        wbase = wid * NW_E

        def fire_idx(k, w):
            pltpu.async_copy(idx_hbm.at[wbase + w], ibuf[k], si[k])

        def wait_idx(k, w):
            pltpu.make_async_copy(idx_hbm.at[wbase + w], ibuf[k],
                                  si[k]).wait()

        def fire_gather(k):
            pltpu.async_copy(a_hbm.at[ibuf[k].at[pl.ds(0, WSZ)]],
                             abuf[k], sg[k])
            pltpu.async_copy(g_hbm.at[ibuf[k].at[pl.ds(WSZ, WSZ)]],
                             gbuf[k], sg[k])

        def wait_gather(k):
            pltpu.make_async_copy(a_hbm.at[ibuf[k].at[pl.ds(0, WSZ)]],
                                  abuf[k], sg[k]).wait()
            pltpu.make_async_copy(g_hbm.at[ibuf[k].at[pl.ds(WSZ, WSZ)]],
                                  gbuf[k], sg[k]).wait()

        def copy_scatter_idx(k):
            def cpb(i, _):
                sibuf[k][pl.ds(i * 16, 16)] = ibuf[k][pl.ds(WSZ + i * 16, 16)]
                return 0
            lax.fori_loop(0, WSZ // 16, cpb, 0, unroll=8)

        def fire_scat(k):
            pltpu.async_copy(mbuf[k], acc.at[sibuf[k]], ss[k], add=True)

        def wait_scat(k):
            pltpu.make_async_copy(mbuf[k], acc.at[sibuf[k]], ss[k]).wait()

        hi_mask = jnp.full((16,), -65536, jnp.int32)

        def compute(k):
            ab, gb, mf = abuf[k], gbuf[k], mbuf[k]

            def cb(i, _):
                wa = ab[i, :]
                wg = gb[i, :]
                a_e = lax.bitcast_convert_type(wa << 16, _f32)
                g_e = lax.bitcast_convert_type(wg << 16, _f32)
                a_o = lax.bitcast_convert_type(wa & hi_mask, _f32)
                g_o = lax.bitcast_convert_type(wg & hi_mask, _f32)
                d0 = a_e - g_e
                d1 = a_o - g_o
                mf[i, pl.ds(0, 16)] = (jnp.maximum(d0, 0.0)
                                       + av * jnp.minimum(d0, 0.0))
                mf[i, pl.ds(16, 16)] = (jnp.maximum(d1, 0.0)
                                        + av * jnp.minimum(d1, 0.0))
                return 0
            lax.fori_loop(0, WSZ, cb, 0, unroll=8)

        fire_idx(0, 0)
        fire_idx(1, 1)

        def body(j, _):
            w0 = 2 * j
            for k in range(2):
                wait_idx(k, w0 + k)

                @pl.when(j > 0)
                def _():
                    wait_scat(k)
                copy_scatter_idx(k)
                fire_gather(k)
            for k in range(2):
                wait_gather(k)

                @pl.when(w0 + k + 2 < NW_E)
                def _():
                    fire_idx(k, w0 + k + 2)
                compute(k)
                fire_scat(k)
            return 0
        lax.fori_loop(0, NW_E // 2, body, 0)
        wait_scat(0)
        wait_scat(1)

        plsc.subcore_barrier()
        pltpu.sync_copy(acc.at[pl.ds(rows0, rows_per_sub)],
                        out_hbm.at[c, pl.ds(rows0, rows_per_sub)])

    return prop


def _make_counts_big():
    rows_per_sub = NPAD // NSUB
    nb = 4

    @functools.partial(
        pl.kernel,
        compiler_params=pltpu.CompilerParams(use_tc_tiling_on_sc=False),
        out_type=jax.ShapeDtypeStruct((NC, NPAD), _f32),
        mesh=_mesh(),
        scratch_types=(
            [pltpu.VMEM((2 * WSZ,), jnp.int32) for _ in range(4)]
            + [pltpu.VMEM((WSZ,), jnp.int32) for _ in range(4)]
            + [pltpu.VMEM((WSZ,), _f32), pltpu.VMEM((WSZ,), _f32)]
            + [pltpu.SemaphoreType.DMA for _ in range(8)]
            + [pltpu.VMEM_SHARED((NPAD,), _f32)]
        ),
    )
    def counts(idx_hbm, out_hbm, *scr):
        ibuf = scr[0:4]
        sibuf = scr[4:8]
        ones, zv = scr[8:10]
        si = scr[10:14]
        ss = scr[14:18]
        acc = scr[18]

        c = lax.axis_index("c")
        s = lax.axis_index("s")
        wid = s * NC + c
        rows0 = s * rows_per_sub

        def ib(i, _):
            ones[pl.ds(i * 16, 16)] = jnp.full((16,), 1.0, _f32)
            zv[pl.ds(i * 16, 16)] = jnp.zeros((16,), _f32)
            return 0
        lax.fori_loop(0, WSZ // 16, ib, 0)

        def zc(k, _):
            pltpu.sync_copy(zv, acc.at[pl.ds(rows0 + k * WSZ, WSZ)])
            return 0
        lax.fori_loop(0, rows_per_sub // WSZ, zc, 0)
        plsc.subcore_barrier()

        wbase = wid * NW_E

        def fire_idx(k, w):
            pltpu.async_copy(idx_hbm.at[wbase + w], ibuf[k], si[k])

        def wait_idx(k, w):
            pltpu.make_async_copy(idx_hbm.at[wbase + w], ibuf[k],
                                  si[k]).wait()

        def copy_scatter_idx(k):
            def cpb(i, _):
                sibuf[k][pl.ds(i * 16, 16)] = ibuf[k][pl.ds(WSZ + i * 16, 16)]
                return 0
            lax.fori_loop(0, WSZ // 16, cpb, 0, unroll=8)

        def fire_scat(k):
            pltpu.async_copy(ones, acc.at[sibuf[k]], ss[k], add=True)

        def wait_scat(k):
            pltpu.make_async_copy(ones, acc.at[sibuf[k]], ss[k]).wait()

        for k in range(nb):
            fire_idx(k, k)

        def body(j, _):
            w0 = nb * j
            for k in range(nb):
                wait_idx(k, w0 + k)

                @pl.when(j > 0)
                def _():
                    wait_scat(k)
                copy_scatter_idx(k)

                @pl.when(w0 + k + nb < NW_E)
                def _():
                    fire_idx(k, w0 + k + nb)
                fire_scat(k)
            return 0
        lax.fori_loop(0, NW_E // nb, body, 0)
        for k in range(nb):
            wait_scat(k)

        plsc.subcore_barrier()
        pltpu.sync_copy(acc.at[pl.ds(rows0, rows_per_sub)],
                        out_hbm.at[c, pl.ds(rows0, rows_per_sub)])

    return counts


def _make_counts(nseg, nwin):
    rows_per_sub = nseg // NSUB

    @functools.partial(
        pl.kernel,
        compiler_params=pltpu.CompilerParams(use_tc_tiling_on_sc=False),
        out_type=jax.ShapeDtypeStruct((NC, nseg), _f32),
        mesh=_mesh(),
        scratch_types=[
            pltpu.VMEM((2, WSZ), jnp.int32),
            pltpu.VMEM((WSZ,), _f32),
            pltpu.VMEM_SHARED((nseg,), _f32),
        ],
    )
    def counts(idx_hbm, out_hbm, idxb, ones, acc):
        c = lax.axis_index("c")
        s = lax.axis_index("s")
        wid = s * NC + c
        rows0 = s * rows_per_sub

        def zb(i, _):
            ones[pl.ds(i * 16, 16)] = jnp.zeros((16,), _f32)
            return 0
        lax.fori_loop(0, WSZ // 16, zb, 0)
        if rows_per_sub <= WSZ:
            pltpu.sync_copy(ones.at[pl.ds(0, rows_per_sub)],
                            acc.at[pl.ds(rows0, rows_per_sub)])
        else:
            def zc(k, _):
                pltpu.sync_copy(ones, acc.at[pl.ds(rows0 + k * WSZ, WSZ)])
                return 0
            lax.fori_loop(0, rows_per_sub // WSZ, zc, 0)

        def ob(i, _):
            ones[pl.ds(i * 16, 16)] = jnp.full((16,), 1.0, _f32)
            return 0
        lax.fori_loop(0, WSZ // 16, ob, 0)
        plsc.subcore_barrier()

        wbase = wid * nwin

        def body(w, _):
            pltpu.sync_copy(idx_hbm.at[wbase + w], idxb)
            pltpu.sync_copy(ones, acc.at[idxb.at[1]], add=True)
            return 0
        lax.fori_loop(0, nwin, body, 0)

        plsc.subcore_barrier()
        pltpu.sync_copy(acc.at[pl.ds(rows0, rows_per_sub)],
                        out_hbm.at[c, pl.ds(rows0, rows_per_sub)])

    return counts


def _make_tab(ntab):
    @functools.partial(
        pl.kernel,
        compiler_params=pltpu.CompilerParams(use_tc_tiling_on_sc=False),
        out_type=jax.ShapeDtypeStruct((NPAD, FT), _f32),
        mesh=_mesh(),
        scratch_types=[
            pltpu.VMEM((WSZ,), jnp.int32),
            pltpu.VMEM((WSZ, FT), _f32),
            pltpu.SemaphoreType.DMA,
        ],
    )
    def tab(t_hbm, idx_hbm, out_hbm, idxb, tbuf, sem):
        c = lax.axis_index("c")
        s = lax.axis_index("s")
        wid = s * NC + c
        ebase = wid * NW_R * WSZ

        def body(w, _):
            pltpu.sync_copy(idx_hbm.at[wid * NW_R + w], idxb)
            pltpu.async_copy(t_hbm.at[idxb], tbuf, sem).wait()
            pltpu.sync_copy(tbuf, out_hbm.at[pl.ds(ebase + w * WSZ, WSZ)])
            return 0
        lax.fori_loop(0, NW_R, body, 0)

    return tab


def _make_emap(nseg):
    @functools.partial(
        pl.kernel,
        compiler_params=pltpu.CompilerParams(use_tc_tiling_on_sc=False),
        out_type=jax.ShapeDtypeStruct((NPAD, F), _f32),
        mesh=_mesh(),
        scratch_types=[
            pltpu.VMEM((2, WSZ), jnp.int32),
            pltpu.VMEM((WSZ, F), _f32),
            pltpu.VMEM((WSZ, F), _f32),
            pltpu.VMEM((16,), _f32),
            pltpu.SemaphoreType.DMA,
            pltpu.SemaphoreType.DMA,
        ],
    )
    def emap(c_hbm, d_hbm, idx_hbm, alpha_hbm, out_hbm,
             idxb, cbuf, dbuf, alphav, sc_, sd_):
        c = lax.axis_index("c")
        s = lax.axis_index("s")
        wid = s * NC + c
        ebase = wid * NW_R * WSZ
        pltpu.sync_copy(alpha_hbm, alphav)
        av = alphav[...]

        def body(w, _):
            pltpu.sync_copy(idx_hbm.at[wid * NW_R + w], idxb)
            pltpu.async_copy(c_hbm.at[idxb.at[0]], cbuf, sc_)
            pltpu.async_copy(d_hbm.at[idxb.at[1]], dbuf, sd_)
            pltpu.make_async_copy(c_hbm.at[idxb.at[0]], cbuf, sc_).wait()
            pltpu.make_async_copy(d_hbm.at[idxb.at[1]], dbuf, sd_).wait()

            def cb(i, _):
                for hh in (0, 16):
                    d = cbuf[i, pl.ds(hh, 16)] + dbuf[i, pl.ds(hh, 16)]
                    cbuf[i, pl.ds(hh, 16)] = (jnp.maximum(d, 0.0)
                                              + av * jnp.minimum(d, 0.0))
                return 0
            lax.fori_loop(0, WSZ, cb, 0, unroll=4)
            pltpu.sync_copy(cbuf, out_hbm.at[pl.ds(ebase + w * WSZ, WSZ)])
            return 0
        lax.fori_loop(0, NW_R, body, 0)

    return emap


def _make_rscatter(nseg):
    rows_per_sub = nseg // NSUB

    @functools.partial(
        pl.kernel,
        compiler_params=pltpu.CompilerParams(use_tc_tiling_on_sc=False),
        out_type=jax.ShapeDtypeStruct((NC, nseg, F), _f32),
        mesh=_mesh(),
        scratch_types=[
            pltpu.VMEM((2, WSZ), jnp.int32),
            pltpu.VMEM((WSZ, F), _f32),
            pltpu.VMEM((WSZ, F), _f32),
            pltpu.VMEM_SHARED((nseg, F), _f32),
            pltpu.SemaphoreType.DMA,
        ],
    )
    def rscatter(m_hbm, idx_hbm, out_hbm, idxb, mbuf, zbuf, acc, sem):
        c = lax.axis_index("c")
        s = lax.axis_index("s")
        wid = s * NC + c
        rows0 = s * rows_per_sub
        ebase = wid * NW_R * WSZ

        def zb(i, _):
            zbuf[i, pl.ds(0, 16)] = jnp.zeros((16,), _f32)
            zbuf[i, pl.ds(16, 16)] = jnp.zeros((16,), _f32)
            return 0
        lax.fori_loop(0, WSZ, zb, 0)
        pltpu.sync_copy(zbuf.at[pl.ds(0, rows_per_sub)],
                        acc.at[pl.ds(rows0, rows_per_sub)])
        plsc.subcore_barrier()

        def body(w, _):
            pltpu.sync_copy(idx_hbm.at[wid * NW_R + w], idxb)
            pltpu.async_copy(m_hbm.at[pl.ds(ebase + w * WSZ, WSZ)],
                             mbuf, sem).wait()
            pltpu.sync_copy(mbuf, acc.at[idxb.at[1]], add=True)
            return 0
        lax.fori_loop(0, NW_R, body, 0)

        plsc.subcore_barrier()
        pltpu.sync_copy(acc.at[pl.ds(rows0, rows_per_sub)],
                        out_hbm.at[c, pl.ds(rows0, rows_per_sub)])

    return rscatter


_PROP = _make_prop()
_COUNTS_BIG = _make_counts_big()
_COUNTS_SRC = _make_counts(NSEG_SRC, NW_R)
_COUNTS_STA = _make_counts(NSEG_STA, NW_R)
_TAB_SRC = _make_tab(NSEG_SRC)
_TAB_STA = _make_tab(NSEG_STA)
_EMAP_SRC = _make_emap(NSEG_SRC)
_EMAP_STA = _make_emap(NSEG_STA)
_RSCAT_SRC = _make_rscatter(NSEG_SRC)
_RSCAT_STA = _make_rscatter(NSEG_STA)


def _prelu(x, a):
    return jnp.where(x >= 0, x, a * x)


def _row_spec(f):
    return pl.BlockSpec((BM, f), lambda i: (i, 0))


def _stack_spec(f):
    return pl.BlockSpec((2, BM, f), lambda i: (0, i, 0))


def _full_spec(shape):
    nd = len(shape)
    return pl.BlockSpec(shape, lambda i: (0,) * nd)


def _smem_spec():
    return pl.BlockSpec(memory_space=pltpu.SMEM)


def _s0_body(al, xp, mk, tg, e0m, e0g, b0, e1, b1, sx, sg, me_o, h0_o):
    a_e = al[0]
    t8 = tg[...][:, 0:8]
    h = _prelu(mk[...] @ e0m[...].T + t8 @ e0g[...].T + b0[...], a_e)
    me_o[...] = h @ e1[...].T + b1[...]
    h0_o[...] = xp[...] @ sx[...].T + t8 @ sg[...].T


def _s1_body(al, h, me, psta, psrc, wih, wim, bi, wh, bme, we3,
             tr_o, g1_o, g2_o, a1_o, a2_o, g1b_o, g2b_o):
    a_init, a11, a12 = al[0], al[1], al[2]
    tr = _prelu(h[...] @ wih[...].T + me[...] @ wim[...].T + bi[...], a_init)
    g1 = psta[...] @ we3[...].T
    g2 = psrc[...] @ we3[...].T
    tr_o[...] = tr
    a1_o[...] = (_prelu(tr, a11) @ wh[...].T + bme[...] + g1).astype(jnp.bfloat16)
    a2_o[...] = (_prelu(tr, a12) @ wh[...].T + bme[...] + g2).astype(jnp.bfloat16)
    g1_o[...] = g1
    g2_o[...] = g2
    g1b_o[...] = g1.astype(jnp.bfloat16)
    g2b_o[...] = g2.astype(jnp.bfloat16)


def _s2_body(al, tr, me, p1, p2, c1, c2, g1, g2,
             wt, wp1, wp2, wm, bb, w21, b21, w22, b22, wh, bme,
             trp_o, a3_o, a4_o):
    a1s, a21, a22 = al[0], al[1], al[2]
    ic1 = 1.0 / jnp.maximum(c1[...][0] + c1[...][1], 1.0)
    ic2 = 1.0 / jnp.maximum(c2[...][0] + c2[...][1], 1.0)
    prop1 = (p1[...][0] + p1[...][1]) * ic1
    prop2 = (p2[...][0] + p2[...][1]) * ic2
    trp = _prelu(tr[...] @ wt[...].T + prop1 @ wp1[...].T
                 + prop2 @ wp2[...].T + me[...] @ wm[...].T + bb[...], a1s)
    u1 = _prelu(trp @ w21[...].T + b21[...], a21)
    u2 = _prelu(trp @ w22[...].T + b22[...], a22)
    trp_o[...] = trp
    a3_o[...] = (u1 @ wh[...].T + bme[...] + g1[...]).astype(jnp.bfloat16)
    a4_o[...] = (u2 @ wh[...].T + bme[...] + g2[...]).astype(jnp.bfloat16)


def _s3_body(al, trp, me, p3, p4, c1, c2, wt, wp1, wp2, wm, bb, h_o):
    a2s = al[0]
    ic1 = 1.0 / jnp.maximum(c1[...][0] + c1[...][1], 1.0)
    ic2 = 1.0 / jnp.maximum(c2[...][0] + c2[...][1], 1.0)
    prop3 = (p3[...][0] + p3[...][1]) * ic1
    prop4 = (p4[...][0] + p4[...][1]) * ic2
    h_o[...] = _prelu(trp[...] @ wt[...].T + prop3 @ wp1[...].T
                      + prop4 @ wp2[...].T + me[...] @ wm[...].T + bb[...], a2s)


def _sr1_body(h, me, psta, psrc, v1, v2, v3, u1, u2, u3, cs_o, ct_o):
    cs_o[...] = h[...] @ v1[...].T + me[...] @ v2[...].T - psta[...] @ v3[...].T
    ct_o[...] = h[...] @ u1[...].T + me[...] @ u2[...].T - psrc[...] @ u3[...].T


def _sr2_body(al, m1, w, b, o):
    o[...] = _prelu(m1[...] @ w[...].T + b[...], al[0])


def _sd_body(srcsp, locsp, v3, b1s, u3, b1t, ds_o, dt_o):
    ds_o[...] = srcsp[...] @ v3[...].T + b1s[...]
    dt_o[...] = locsp[...] @ u3[...].T + b1t[...]


def _sf_body(al, ssrc, cs, ssta, ct, memp,
             fw2s, fb2s, fw2t, fb2t, pmw0, pmb0, pmw1, pmb1,
             mdw0a, mdw0b, mdb0, mdw1, mdb1,
             pw0, pb0, pw1, pb1, tw0, tb0, tw1, tb1, cw0, cb0, cw1, cb1,
             pred_o, predt_o, corr_o):
    a2s, a2t, apm, amd, apj, apt, apc = (al[0], al[1], al[2], al[3],
                                         al[4], al[5], al[6])
    invs = 1.0 / jnp.maximum(cs[...][0] + cs[...][1], 1.0)
    aggs = (ssrc[...][0] + ssrc[...][1]) * invs
    semb = _prelu(aggs @ fw2s[...].T + fb2s[...], a2s)
    mp = _prelu(memp[...] @ pmw0[...].T + pmb0[...], apm) @ pmw1[...].T + pmb1[...]
    mer = (_prelu(semb @ mdw0a[...].T + mp @ mdw0b[...].T + mdb0[...], amd)
           @ mdw1[...].T + mdb1[...])
    pred_o[...] = (_prelu(mer @ pw0[...].T + pb0[...], apj)
                   @ pw1[...].T + pb1[...]) * 5000.0
    predt_o[...] = (_prelu(mer @ tw0[...].T + tb0[...], apt)
                    @ tw1[...].T + tb1[...])
    invt = 1.0 / jnp.maximum(ct[...][0] + ct[...][1], 1.0)
    aggt = (ssta[...][0] + ssta[...][1]) * invt
    temb = _prelu(aggt @ fw2t[...].T + fb2t[...], a2t)
    corr_o[...] = (_prelu(temb @ cw0[...].T + cb0[...], apc)
                   @ cw1[...].T + cb1[...])


def _shape(n, f):
    return jax.ShapeDtypeStruct((n, f), _f32)


_S0 = pl.pallas_call(
    _s0_body,
    grid=(GRID,),
    in_specs=[_smem_spec(), _row_spec(24), _row_spec(24), _row_spec(FT),
              _full_spec((32, 24)), _full_spec((32, 8)), _full_spec((1, 32)),
              _full_spec((16, 32)), _full_spec((1, 16)),
              _full_spec((32, 24)), _full_spec((32, 8))],
    out_specs=[_row_spec(FT), _row_spec(F)],
    out_shape=[_shape(NPAD, FT), _shape(NPAD, F)],
)

_S1 = pl.pallas_call(
    _s1_body,
    grid=(GRID,),
    in_specs=[_smem_spec(), _row_spec(F), _row_spec(FT), _row_spec(4),
              _row_spec(4),
              _full_spec((F, F)), _full_spec((F, FT)), _full_spec((1, F)),
              _full_spec((F, F)), _full_spec((1, F)), _full_spec((F, 4))],
    out_specs=[_row_spec(F)] * 7,
    out_shape=[_shape(NPAD, F)] * 3
    + [jax.ShapeDtypeStruct((NPAD, F), jnp.bfloat16)] * 4,
)

_S2 = pl.pallas_call(
    _s2_body,
    grid=(GRID,),
    in_specs=[_smem_spec(), _row_spec(F), _row_spec(FT),
              _stack_spec(F), _stack_spec(F),
              _stack_spec(1), _stack_spec(1),
              _row_spec(F), _row_spec(F),
              _full_spec((64, F)), _full_spec((64, F)), _full_spec((64, F)),
              _full_spec((64, FT)), _full_spec((1, 64)),
              _full_spec((F, 64)), _full_spec((1, F)),
              _full_spec((F, 64)), _full_spec((1, F)),
              _full_spec((F, F)), _full_spec((1, F))],
    out_specs=[_row_spec(64), _row_spec(F), _row_spec(F)],
    out_shape=[_shape(NPAD, 64),
               jax.ShapeDtypeStruct((NPAD, F), jnp.bfloat16),
               jax.ShapeDtypeStruct((NPAD, F), jnp.bfloat16)],
)

_S3 = pl.pallas_call(
    _s3_body,
    grid=(GRID,),
    in_specs=[_smem_spec(), _row_spec(64), _row_spec(FT),
              _stack_spec(F), _stack_spec(F),
              _stack_spec(1), _stack_spec(1),
              _full_spec((F, 64)), _full_spec((F, F)), _full_spec((F, F)),
              _full_spec((F, FT)), _full_spec((1, F))],
    out_specs=[_row_spec(F)],
    out_shape=[_shape(NPAD, F)],
)

_SR1 = pl.pallas_call(
    _sr1_body,
    grid=(GRID,),
    in_specs=[_row_spec(F), _row_spec(FT), _row_spec(4), _row_spec(4),
              _full_spec((F, F)), _full_spec((F, FT)), _full_spec((F, 4)),
              _full_spec((F, F)), _full_spec((F, FT)), _full_spec((F, 4))],
    out_specs=[_row_spec(F), _row_spec(F)],
    out_shape=[_shape(NPAD, F), _shape(NPAD, F)],
)

_SR2 = pl.pallas_call(
    _sr2_body,
    grid=(GRID,),
    in_specs=[_smem_spec(), _row_spec(F),
              _full_spec((F, F)), _full_spec((1, F))],
    out_specs=[_row_spec(F)],
    out_shape=[_shape(NPAD, F)],
)

_SD = pl.pallas_call(
    _sd_body,
    grid=(1,),
    in_specs=[_full_spec((NSEG_SRC, 4)), _full_spec((NSEG_STA, 4)),
              _full_spec((F, 4)), _full_spec((1, F)),
              _full_spec((F, 4)), _full_spec((1, F))],
    out_specs=[_full_spec((NSEG_SRC, F)), _full_spec((NSEG_STA, F))],
    out_shape=[_shape(NSEG_SRC, F), _shape(NSEG_STA, F)],
)

_SF = pl.pallas_call(
    _sf_body,
    grid=(1,),
    in_specs=[_smem_spec(),
              _full_spec((NC, NSEG_SRC, F)), _full_spec((NC, NSEG_SRC, 1)),
              _full_spec((NC, NSEG_STA, F)), _full_spec((NC, NSEG_STA, 1)),
              _full_spec((NSEG_SRC, 8)),
              _full_spec((16, F)), _full_spec((1, 16)),
              _full_spec((16, F)), _full_spec((1, 16)),
              _full_spec((F, 8)), _full_spec((1, F)),
              _full_spec((16, F)), _full_spec((1, 16)),
              _full_spec((F, 16)), _full_spec((F, 16)), _full_spec((1, F)),
              _full_spec((F, F)), _full_spec((1, F)),
              _full_spec((F, F)), _full_spec((1, F)),
              _full_spec((8, F)), _full_spec((1, 8)),
              _full_spec((16, F)), _full_spec((1, 16)),
              _full_spec((8, 16)), _full_spec((1, 8)),
              _full_spec((16, 16)), _full_spec((1, 16)),
              _full_spec((8, 16)), _full_spec((1, 8))],
    out_specs=[_full_spec((NSEG_SRC, 8)), _full_spec((NSEG_SRC, 8)),
               _full_spec((NSEG_STA, 8))],
    out_shape=[_shape(NSEG_SRC, 8), _shape(NSEG_SRC, 8), _shape(NSEG_STA, 8)],
)


_PERM = tuple(list(range(0, F, 2)) + list(range(1, F, 2)))


def _i32rows(x):
    return jax.lax.bitcast_convert_type(
        x.reshape(NPAD, F // 2, 2), jnp.int32)


def _pad2(w, r, c):
    return jnp.zeros((r, c), _f32).at[:w.shape[0], :w.shape[1]].set(w)


def _padb(b, c):
    return jnp.zeros((1, c), _f32).at[0, :b.shape[0]].set(b)


def _pad_nodes(x, cols):
    return jnp.zeros((NPAD, cols), _f32).at[:x.shape[0], :x.shape[1]].set(x)


def _prep_edges(src, dst, nwin, pad_base, pad_mod):
    ne = src.shape[0]
    npad = NWRK * nwin * WSZ - ne
    srcp = jnp.concatenate([src, jnp.zeros((npad,), jnp.int32)])
    dstp = jnp.concatenate(
        [dst, pad_base + (jnp.arange(npad, dtype=jnp.int32) % pad_mod)])
    idx = jnp.stack([srcp.reshape(NWRK, nwin, WSZ),
                     dstp.reshape(NWRK, nwin, WSZ)], axis=2)
    return idx.reshape(NWRK * nwin, 2, WSZ)


def _prep_edges_big(src, dst, pad_base, pad_mod):
    ne = src.shape[0]
    npad = EPAD - ne
    srcp = jnp.concatenate([src, jnp.zeros((npad,), jnp.int32)])
    dstp = jnp.concatenate(
        [dst, pad_base + (jnp.arange(npad, dtype=jnp.int32) % pad_mod)])
    idx = jnp.stack([srcp.reshape(NWRK, NW_E, WSZ),
                     dstp.reshape(NWRK, NW_E, WSZ)], axis=2)
    return idx.reshape(NWRK * NW_E, 2 * WSZ)


def _prep_node_idx(idx):
    p = jnp.zeros((NPAD,), jnp.int32).at[:idx.shape[0]].set(idx)
    return p.reshape(NWRK * NW_R, WSZ)


def _avec(a):
    return jnp.full((16,), a, _f32)


def kernel(x, mask, A_in_pick, A_in_src, A_src_in_product, A_sta_in_product,
           A_src_in_sta, locs_cart, srcs_cart, memory, params):
    sta_id = A_src_in_sta[0]
    src_id = A_src_in_sta[1]

    t_src = jnp.zeros((NSEG_SRC, FT), _f32)
    t_src = t_src.at[:memory.shape[0], 0:4].set(memory)
    t_src = t_src.at[:srcs_cart.shape[0], 4:7].set(srcs_cart / 30000.0)
    t_sta = jnp.zeros((NSEG_STA, FT), _f32)
    t_sta = t_sta.at[:locs_cart.shape[0], 0:3].set(locs_cart / 30000.0)

    idx_src_nodes = _prep_node_idx(src_id)
    idx_sta_nodes = _prep_node_idx(sta_id)
    idx_pick = _prep_edges_big(A_in_pick[0], A_in_pick[1], NREAL, 1024)
    idx_srce = _prep_edges_big(A_in_src[0], A_in_src[1], NREAL, 1024)
    idx_rsrc = _prep_edges(A_src_in_product[0], A_src_in_product[1], NW_R,
                           1000, 24)
    idx_rsta = _prep_edges(A_sta_in_product[0], A_sta_in_product[1], NW_R,
                           500, 12)

    xp = _pad_nodes(x, 24)
    maskp = _pad_nodes(mask, 24)

    tg_src = _TAB_SRC(t_src, idx_src_nodes)
    tg_sta = _TAB_STA(t_sta, idx_sta_nodes)
    cnt_pick = _COUNTS_BIG(idx_pick).reshape(NC, NPAD, 1)
    cnt_srce = _COUNTS_BIG(idx_srce).reshape(NC, NPAD, 1)
    cnt_rsrc = _COUNTS_SRC(idx_rsrc).reshape(NC, NSEG_SRC, 1)
    cnt_rsta = _COUNTS_STA(idx_rsta).reshape(NC, NSEG_STA, 1)

    psta4 = tg_sta[:, 0:4]
    psrc4 = tg_src[:, 4:8]

    pe = params['embed_inpt']
    e0 = pe['layers'][0]['W']
    e1 = pe['layers'][1]['W']
    sx = jnp.zeros((F, 24), _f32).at[:18, :18].set(jnp.eye(18))
    sg = jnp.zeros((F, 8), _f32).at[18:22, 0:4].set(jnp.eye(4))
    al0 = jnp.array([pe['a']] + [0.0] * 7, _f32)
    mask_e, h = _S0(al0, xp, maskp, tg_src,
                    _pad2(e0[:, :18], 32, 24), _pad2(e0[:, 18:22], 32, 8),
                    _padb(pe['layers'][0]['b'], 32),
                    _pad2(e1, 16, 32), _padb(pe['layers'][1]['b'], 16),
                    sx, sg)

    for name in ['da1', 'da2', 'da3', 'da4', 'da5']:
        p = params[name]
        We = p['merge_edges']['W']
        wh = _pad2(We[:, :30], F, F)
        we3 = _pad2(We[:, 30:33], F, 4)
        bme = _padb(p['merge_edges']['b'], F)
        nin = 22 if name == 'da1' else 30
        wi = p['init_trns']['W']
        al1 = jnp.array([p['a_init'], p['a11'], p['a12'], 0, 0, 0, 0, 0], _f32)
        tr, g1, g2, a1, a2, g1b, g2b = _S1(
            al1, h, mask_e, psta4, psrc4,
            _pad2(wi[:, :nin], F, F), _pad2(wi[:, nin:nin + 10], F, FT),
            _padb(p['init_trns']['b'], F), wh, bme, we3)

        am = _avec(p['a_merge'])
        p1 = _PROP(_i32rows(a1), _i32rows(g1b), idx_pick, am)
        p2 = _PROP(_i32rows(a2), _i32rows(g2b), idx_srce, am)

        w11, w12 = p['l1_t1_2']['W'], p['l1_t2_2']['W']
        wt = jnp.zeros((64, F), _f32)
        wt = wt.at[0:30, 0:30].set(w11[:, 0:30]).at[30:60, 0:30].set(w12[:, 0:30])
        wp1 = jnp.zeros((64, F), _f32).at[0:30, 0:30].set(w11[:, 30:60])[:, _PERM]
        wp2 = jnp.zeros((64, F), _f32).at[30:60, 0:30].set(w12[:, 30:60])[:, _PERM]
        wm = jnp.zeros((64, FT), _f32)
        wm = wm.at[0:30, 0:10].set(w11[:, 60:70]).at[30:60, 0:10].set(w12[:, 60:70])
        bb = jnp.zeros((1, 64), _f32)
        bb = bb.at[0, 0:30].set(p['l1_t1_2']['b']).at[0, 30:60].set(p['l1_t2_2']['b'])
        al2 = jnp.array([p['a1'], p['a21'], p['a22'], 0, 0, 0, 0, 0], _f32)
        trp, a3, a4 = _S2(
            al2, tr, mask_e, p1, p2, cnt_pick, cnt_srce, g1, g2,
            wt, wp1, wp2, wm, bb,
            _pad2(p['l2_t1_1']['W'], F, 64), _padb(p['l2_t1_1']['b'], F),
            _pad2(p['l2_t2_1']['W'], F, 64), _padb(p['l2_t2_1']['b'], F),
            wh, bme)

        p3 = _PROP(_i32rows(a3), _i32rows(g1b), idx_pick, am)
        p4 = _PROP(_i32rows(a4), _i32rows(g2b), idx_srce, am)

        w21, w22 = p['l2_t1_2']['W'], p['l2_t2_2']['W']
        wt2 = jnp.zeros((F, 64), _f32)
        wt2 = wt2.at[0:15, 0:60].set(w21[:, 0:60]).at[15:30, 0:60].set(w22[:, 0:60])
        wp3 = jnp.zeros((F, F), _f32).at[0:15, 0:30].set(w21[:, 60:90])[:, _PERM]
        wp4 = jnp.zeros((F, F), _f32).at[15:30, 0:30].set(w22[:, 60:90])[:, _PERM]
        wm2 = jnp.zeros((F, FT), _f32)
        wm2 = wm2.at[0:15, 0:10].set(w21[:, 90:100]).at[15:30, 0:10].set(w22[:, 90:100])
        bb2 = jnp.zeros((1, F), _f32)
        bb2 = bb2.at[0, 0:15].set(p['l2_t1_2']['b']).at[0, 15:30].set(p['l2_t2_2']['b'])
        al3 = jnp.array([p['a2'], 0, 0, 0, 0, 0, 0, 0], _f32)
        (h,) = _S3(al3, trp, mask_e, p3, p4, cnt_pick, cnt_srce,
                   wt2, wp3, wp4, wm2, bb2)

    pbs, pbt = params['bip_src'], params['bip_sta']
    v_w, u_w = pbs['fc1_0']['W'], pbt['fc1_0']['W']
    c_src, c_sta = _SR1(
        h, mask_e, psta4, psrc4,
        _pad2(v_w[:, :30], F, F), _pad2(v_w[:, 30:40], F, FT),
        _pad2(v_w[:, 40:43], F, 4),
        _pad2(u_w[:, :30], F, F), _pad2(u_w[:, 30:40], F, FT),
        _pad2(u_w[:, 40:43], F, 4))

    srcsp = jnp.zeros((NSEG_SRC, 4), _f32).at[:srcs_cart.shape[0], 0:3].set(
        srcs_cart / 30000.0)
    locsp = jnp.zeros((NSEG_STA, 4), _f32).at[:locs_cart.shape[0], 0:3].set(
        locs_cart / 30000.0)
    d_src, d_sta = _SD(srcsp, locsp,
                       _pad2(v_w[:, 40:43], F, 4), _padb(pbs['fc1_0']['b'], F),
                       _pad2(u_w[:, 40:43], F, 4), _padb(pbt['fc1_0']['b'], F))

    m1_src = _EMAP_SRC(c_src, d_src, idx_rsrc, _avec(pbs['a_fc1']))
    m1_sta = _EMAP_STA(c_sta, d_sta, idx_rsta, _avec(pbt['a_fc1']))

    als = jnp.array([pbs['a1'], 0, 0, 0, 0, 0, 0, 0], _f32)
    alt = jnp.array([pbt['a1'], 0, 0, 0, 0, 0, 0, 0], _f32)
    (m2_src,) = _SR2(als, m1_src, _pad2(pbs['fc1_2']['W'], F, F),
                     _padb(pbs['fc1_2']['b'], F))
    (m2_sta,) = _SR2(alt, m1_sta, _pad2(pbt['fc1_2']['W'], F, F),
                     _padb(pbt['fc1_2']['b'], F))

    s_src = _RSCAT_SRC(m2_src, idx_rsrc)
    s_sta = _RSCAT_STA(m2_sta, idx_rsta)

    pm, md = params['proj_memory'], params['merge_data']
    pj, pt, pc = params['proj'], params['proj_t'], params['proj_c']
    memp = jnp.zeros((NSEG_SRC, 8), _f32).at[:memory.shape[0], 0:4].set(memory)
    alf = jnp.array([pbs['a2'], pbt['a2'], pm['a'], md['a'],
                     pj['a'], pt['a'], pc['a'], 0.0], _f32)
    md0 = md['layers'][0]['W']
    pred, pred_t, corr = _SF(
        alf, s_src, cnt_rsrc, s_sta, cnt_rsta, memp,
        _pad2(pbs['fc2']['W'], 16, F), _padb(pbs['fc2']['b'], 16),
        _pad2(pbt['fc2']['W'], 16, F), _padb(pbt['fc2']['b'], 16),
        _pad2(pm['layers'][0]['W'], F, 8), _padb(pm['layers'][0]['b'], F),
        _pad2(pm['layers'][1]['W'], 16, F), _padb(pm['layers'][1]['b'], 16),
        _pad2(md0[:, :15], F, 16), _pad2(md0[:, 15:30], F, 16),
        _padb(md['layers'][0]['b'], F),
        _pad2(md['layers'][1]['W'], F, F), _padb(md['layers'][1]['b'], F),
        _pad2(pj['layers'][0]['W'], F, F), _padb(pj['layers'][0]['b'], F),
        _pad2(pj['layers'][1]['W'], 8, F), _padb(pj['layers'][1]['b'], 8),
        _pad2(pt['layers'][0]['W'], 16, F), _padb(pt['layers'][0]['b'], 16),
        _pad2(pt['layers'][1]['W'], 8, 16), _padb(pt['layers'][1]['b'], 8),
        _pad2(pc['layers'][0]['W'], 16, 16), _padb(pc['layers'][0]['b'], 16),
        _pad2(pc['layers'][1]['W'], 8, 16), _padb(pc['layers'][1]['b'], 8))

    return (pred[:1000, :3], pred_t[:1000, :1], corr[:500, :2])
